# baseline (speedup 1.0000x reference)
import functools

import jax
import jax.numpy as jnp
from jax import lax
from jax.experimental import pallas as pl
from jax.experimental.pallas import tpu as pltpu
from jax.experimental.pallas import tpu_sc as plsc

H = 128
RBF_K = 16
POS_EMB = 16
EB = 3200
BN = 2048
NP = 10240
NSL = NP // 16
NWORK = 32
RC = 80
SCH = 2000

_f32 = jnp.float32
_i32 = jnp.int32


def _mesh():
    return plsc.VectorSubcoreMesh(core_axis_name="c", subcore_axis_name="s")


def _wid():
    return lax.axis_index("s") * 2 + lax.axis_index("c")


def _mm_t(a, b):
    return lax.dot_general(a, b, (((1,), (1,)), ((), ())),
                           preferred_element_type=_f32)


def _make_prologue(E, N):
    TPE = E // NWORK
    NCH = TPE // RC

    @functools.partial(
        pl.kernel,
        out_type=[jax.ShapeDtypeStruct((4 * E,), _f32),
                  jax.ShapeDtypeStruct((48 * E,), _f32)],
        mesh=_mesh(),
        compiler_params=pltpu.CompilerParams(needs_layout_passes=False),
        scratch_types=[pltpu.VMEM((N * 4,), _f32),
                       pltpu.VMEM((512 * 48,), _f32),
                       pltpu.VMEM((RC,), _i32),
                       pltpu.VMEM((RC,), _i32),
                       pltpu.VMEM((4 * RC,), _f32),
                       pltpu.VMEM((48 * RC,), _f32)],
    )
    def k(posf, srcr, dstr, pef, vec_o, pf_o,
          posv, pev, sbuf, dbuf, vecb, pfb):
        base = _wid() * TPE
        pltpu.sync_copy(posf, posv)
        pltpu.sync_copy(pef, pev)

        def chunk(kk, _):
            off = base + kk * RC
            pltpu.sync_copy(srcr.at[pl.ds(off, RC)], sbuf)
            pltpu.sync_copy(dstr.at[pl.ds(off, RC)], dbuf)

            def vr(i, _):
                sl = pl.ds(i * 16, 16)
                s16 = sbuf[sl]
                d16 = dbuf[sl]
                s4 = s16 * 4
                d4 = d16 * 4
                for c in range(3):
                    a = plsc.load_gather(posv, [s4 + c])
                    b = plsc.load_gather(posv, [d4 + c])
                    vecb[pl.ds(c * RC + i * 16, 16)] = a - b
                vecb[pl.ds(3 * RC + i * 16, 16)] = jnp.zeros((16,), _f32)
                rel48 = jnp.clip(d16 - s16, 0, 511) * 48
                for j in range(48):
                    pfb[pl.ds(j * RC + i * 16, 16)] = plsc.load_gather(
                        pev, [rel48 + j])
                return 0

            lax.fori_loop(0, RC // 16, vr, 0)
            for c in range(4):
                pltpu.sync_copy(vecb.at[pl.ds(c * RC, RC)],
                                vec_o.at[pl.ds(c * E + off, RC)])
            for j in range(48):
                pltpu.sync_copy(pfb.at[pl.ds(j * RC, RC)],
                                pf_o.at[pl.ds(j * E + off, RC)])
            return 0

        lax.fori_loop(0, NCH, chunk, 0)

    return k


def _make_gather_rows(E, N):
    TPE = E // NWORK
    NCH = TPE // RC

    @functools.partial(
        pl.kernel,
        out_type=jax.ShapeDtypeStruct((E, H), _f32),
        mesh=_mesh(),
        compiler_params=pltpu.CompilerParams(needs_layout_passes=False),
        scratch_types=[pltpu.VMEM((RC,), _i32),
                       pltpu.VMEM((RC, H), _f32),
                       pltpu.SemaphoreType.DMA],
    )
    def k(hp, srcr, out, ibuf, rbuf, sem):
        base = _wid() * TPE

        def chunk(kk, _):
            off = base + kk * RC
            pltpu.sync_copy(srcr.at[pl.ds(off, RC)], ibuf)
            pltpu.async_copy(hp.at[ibuf], rbuf, sem).wait()
            pltpu.sync_copy(rbuf, out.at[pl.ds(off, RC)])
            return 0

        lax.fori_loop(0, NCH, chunk, 0)

    return k


def _make_segmax(E):
    TPE = E // NWORK
    NCH = TPE // SCH

    @functools.partial(
        pl.kernel,
        out_type=jax.ShapeDtypeStruct((2 * NP,), _f32),
        mesh=_mesh(),
        compiler_params=pltpu.CompilerParams(needs_layout_passes=False),
        scratch_types=[pltpu.VMEM((NP,), _f32),
                       pltpu.VMEM((SCH,), _i32),
                       pltpu.VMEM((SCH,), _f32),
                       pltpu.VMEM((NSL,), _f32),
                       pltpu.VMEM((NSL,), _f32),
                       pltpu.VMEM_SHARED((16 * NP,), _f32)],
    )
    def k(score, dstr, m2, mloc, dbuf, sbuf, acc, tmp, shm):
        cid = lax.axis_index("c")
        sid = lax.axis_index("s")
        base = _wid() * TPE
        iota = lax.iota(_i32, 16)

        def z(i, _):
            mloc[pl.ds(i * 16, 16)] = jnp.full((16,), -1e30, _f32)
            return 0

        lax.fori_loop(0, NP // 16, z, 0)

        def chunk(kk, _):
            off = base + kk * SCH
            pltpu.sync_copy(dstr.at[pl.ds(off, SCH)], dbuf)
            pltpu.sync_copy(score.at[pl.ds(off, SCH)], sbuf)

            def vr(i, _):
                d0 = dbuf[pl.ds(i * 16, 16)]
                s0 = sbuf[pl.ds(i * 16, 16)]
                d, s = plsc.sort_key_val(d0, s0)
                run = s
                for ksh in (1, 2, 4, 8):
                    idx = jnp.maximum(iota - ksh, 0)
                    dsh = d.at[idx].get(mode="promise_in_bounds")
                    rsh = run.at[idx].get(mode="promise_in_bounds")
                    ok = (iota >= ksh) & (dsh == d)
                    run = jnp.where(ok, jnp.maximum(run, rsh), run)
                nxt = jnp.minimum(iota + 1, 15)
                dn = d.at[nxt].get(mode="promise_in_bounds")
                last = (dn != d) | (iota == 15)
                cur = plsc.load_gather(mloc, [d])
                plsc.store_scatter(mloc, [d], jnp.maximum(cur, run),
                                   mask=last)
                return 0

            lax.fori_loop(0, SCH // 16, vr, 0)
            return 0

        lax.fori_loop(0, NCH, chunk, 0)

        pltpu.sync_copy(mloc, shm.at[pl.ds(sid * NP, NP)])
        plsc.subcore_barrier()
        s0 = sid * NSL
        pltpu.sync_copy(shm.at[pl.ds(s0, NSL)], acc)
        for r in range(1, 16):
            pltpu.sync_copy(shm.at[pl.ds(r * NP + s0, NSL)], tmp)

            def mx(i, _):
                acc[pl.ds(i * 16, 16)] = jnp.maximum(
                    acc[pl.ds(i * 16, 16)], tmp[pl.ds(i * 16, 16)])
                return 0

            lax.fori_loop(0, NSL // 16, mx, 0)
        pltpu.sync_copy(acc, m2.at[pl.ds(cid * NP + s0, NSL)])

    return k


def _make_segden(E):
    TPE = E // NWORK
    NCH = TPE // SCH

    @functools.partial(
        pl.kernel,
        out_type=[jax.ShapeDtypeStruct((2 * NP,), _f32),
                  jax.ShapeDtypeStruct((E,), _f32)],
        mesh=_mesh(),
        compiler_params=pltpu.CompilerParams(needs_layout_passes=False),
        scratch_types=[pltpu.VMEM((NP,), _f32),
                       pltpu.VMEM((NP,), _f32),
                       pltpu.VMEM((NP,), _f32),
                       pltpu.VMEM((SCH,), _i32),
                       pltpu.VMEM((SCH,), _f32),
                       pltpu.VMEM((NSL,), _f32),
                       pltpu.VMEM((NSL,), _f32),
                       pltpu.VMEM_SHARED((16 * NP,), _f32)],
    )
    def k(m2, score, dstr, den2, w_o, ma, mb, den, dbuf, sbuf, acc, tmp,
          shm):
        cid = lax.axis_index("c")
        sid = lax.axis_index("s")
        base = _wid() * TPE
        pltpu.sync_copy(m2.at[pl.ds(0, NP)], ma)
        pltpu.sync_copy(m2.at[pl.ds(NP, NP)], mb)

        def mmax(i, _):
            ma[pl.ds(i * 16, 16)] = jnp.maximum(ma[pl.ds(i * 16, 16)],
                                                mb[pl.ds(i * 16, 16)])
            return 0

        lax.fori_loop(0, NP // 16, mmax, 0)

        def z(i, _):
            den[pl.ds(i * 16, 16)] = jnp.zeros((16,), _f32)
            return 0

        lax.fori_loop(0, NP // 16, z, 0)

        def chunk(kk, _):
            off = base + kk * SCH
            pltpu.sync_copy(dstr.at[pl.ds(off, SCH)], dbuf)
            pltpu.sync_copy(score.at[pl.ds(off, SCH)], sbuf)

            def vr(i, _):
                d = dbuf[pl.ds(i * 16, 16)]
                s = sbuf[pl.ds(i * 16, 16)]
                mv = plsc.load_gather(ma, [d])
                ex = jnp.exp(s - mv)
                sbuf[pl.ds(i * 16, 16)] = ex
                plsc.addupdate_scatter(den, [d], ex)
                return 0

            lax.fori_loop(0, SCH // 16, vr, 0)
            pltpu.sync_copy(sbuf, w_o.at[pl.ds(off, SCH)])
            return 0

        lax.fori_loop(0, NCH, chunk, 0)

        pltpu.sync_copy(den, shm.at[pl.ds(sid * NP, NP)])
        plsc.subcore_barrier()
        s0 = sid * NSL
        pltpu.sync_copy(shm.at[pl.ds(s0, NSL)], acc)
        for r in range(1, 16):
            pltpu.sync_copy(shm.at[pl.ds(r * NP + s0, NSL)], tmp)

            def ad(i, _):
                acc[pl.ds(i * 16, 16)] = (acc[pl.ds(i * 16, 16)]
                                          + tmp[pl.ds(i * 16, 16)])
                return 0

            lax.fori_loop(0, NSL // 16, ad, 0)
        pltpu.sync_copy(acc, den2.at[pl.ds(cid * NP + s0, NSL)])

    return k


def _make_segagg(E):
    TPE = E // NWORK
    NCH = TPE // RC
    ZR = 64

    @functools.partial(
        pl.kernel,
        out_type=jax.ShapeDtypeStruct((2 * NP, H), _f32),
        mesh=_mesh(),
        compiler_params=pltpu.CompilerParams(needs_layout_passes=False),
        scratch_types=[pltpu.VMEM((RC,), _i32),
                       pltpu.VMEM((RC,), _f32),
                       pltpu.VMEM((RC, H), _f32),
                       pltpu.VMEM((ZR, H), _f32),
                       pltpu.VMEM_SHARED((NP, H), _f32)],
    )
    def k(w, dstr, t, aggr2, idxb, wb, tb, zb, ash):
        cid = lax.axis_index("c")
        sid = lax.axis_index("s")
        base = _wid() * TPE

        def zz(i, _):
            for j in range(H // 16):
                zb[i, pl.ds(j * 16, 16)] = jnp.zeros((16,), _f32)
            return 0

        lax.fori_loop(0, ZR, zz, 0)
        s0 = sid * NSL
        for j in range(NSL // ZR):
            pltpu.sync_copy(zb, ash.at[pl.ds(s0 + j * ZR, ZR)])
        plsc.subcore_barrier()

        def chunk(kk, _):
            off = base + kk * RC
            pltpu.sync_copy(dstr.at[pl.ds(off, RC)], idxb)
            pltpu.sync_copy(w.at[pl.ds(off, RC)], wb)
            pltpu.sync_copy(t.at[pl.ds(off, RC)], tb)

            def row(e, _):
                ws = plsc.load_gather(wb, [jnp.full((16,), e, _i32)])
                for j in range(H // 16):
                    tb[e, pl.ds(j * 16, 16)] = tb[e, pl.ds(j * 16, 16)] * ws
                return 0

            lax.fori_loop(0, RC, row, 0)
            pltpu.sync_copy(tb, ash.at[idxb], add=True)
            return 0

        lax.fori_loop(0, NCH, chunk, 0)
        plsc.subcore_barrier()
        pltpu.sync_copy(ash.at[pl.ds(s0, NSL)],
                        aggr2.at[pl.ds(cid * NP + s0, NSL)])

    return k


def _edge_body(hs_ref, vec_ref, pf_ref, rbfw_ref, w1r_ref, w1p_ref,
               w1d_ref, b1_ref, w2_ref, b2_ref, t_ref, score_ref):
    v4 = vec_ref[...]
    d2r = jnp.sum(v4 * v4, axis=0, keepdims=True)
    dcol = lax.dot_general(d2r, jnp.ones((1, 1), _f32),
                           (((0,), (1,)), ((), ())),
                           preferred_element_type=_f32)
    dist_col = jnp.sqrt(dcol)
    rbf = jnp.exp(-jnp.square(_mm_t(dist_col, rbfw_ref[...])))
    invdr = 1.0 / jnp.clip(jnp.sqrt(d2r), 1e-12, None)
    dir4 = v4 * invdr
    f_dir = lax.dot_general(dir4, w1d_ref[...],
                            (((0,), (1,)), ((), ())),
                            preferred_element_type=_f32)
    f_pos = lax.dot_general(pf_ref[...], w1p_ref[...],
                            (((0,), (1,)), ((), ())),
                            preferred_element_type=_f32)
    f1 = _mm_t(rbf, w1r_ref[...]) + f_pos + f_dir + b1_ref[...]
    f = _mm_t(jnp.maximum(f1, 0.0), w2_ref[...]) + b2_ref[...]
    hs = hs_ref[...]
    score_ref[...] = jnp.sum(hs * f, axis=1, keepdims=True)
    t_ref[...] = hs + f


def _edge_pass(hsrc, vecT, pfT, p, lidx):
    E = hsrc.shape[0]
    rbfw = p["rbf_W"].reshape(RBF_K, 1)
    w1 = p["frame_W1"]
    w1r = w1[:, :RBF_K]
    w1p = w1[:, RBF_K:RBF_K + POS_EMB]
    w1d = jnp.pad(w1[:, RBF_K + POS_EMB:], ((0, 0), (0, 1)))
    b1 = p["frame_b1"].reshape(1, H)
    w2 = p["frame_W2"]
    b2 = p["frame_b2"].reshape(1, H)
    grid = E // EB
    t, score = pl.pallas_call(
        _edge_body,
        grid=(grid,),
        in_specs=[
            pl.BlockSpec((EB, H), lambda i: (i, 0)),
            pl.BlockSpec((4, EB), lambda i: (0, i)),
            pl.BlockSpec((POS_EMB, EB), lambda i, _l=lidx: (_l, i)),
            pl.BlockSpec((RBF_K, 1), lambda i: (0, 0)),
            pl.BlockSpec((H, RBF_K), lambda i: (0, 0)),
            pl.BlockSpec((H, POS_EMB), lambda i: (0, 0)),
            pl.BlockSpec((H, 4), lambda i: (0, 0)),
            pl.BlockSpec((1, H), lambda i: (0, 0)),
            pl.BlockSpec((H, H), lambda i: (0, 0)),
            pl.BlockSpec((1, H), lambda i: (0, 0)),
        ],
        out_specs=[
            pl.BlockSpec((EB, H), lambda i: (i, 0)),
            pl.BlockSpec((EB, 1), lambda i: (i, 0)),
        ],
        out_shape=[
            jax.ShapeDtypeStruct((E, H), _f32),
            jax.ShapeDtypeStruct((E, 1), _f32),
        ],
    )(hsrc, vecT, pfT, rbfw, w1r, w1p, w1d, b1, w2, b2)
    return t, score.reshape(E)


def _nk0_body(x_ref, w_ref, b_ref, xw_ref, xb_ref, h_ref, hp_ref):
    hv = _mm_t(x_ref[...], w_ref[...]) + b_ref[...]
    h_ref[...] = hv
    hp_ref[...] = _mm_t(hv, xw_ref[...]) + xb_ref[...]


def _nk0(x, params):
    N = x.shape[0]
    p0 = params["layers"][0]
    return pl.pallas_call(
        _nk0_body,
        grid=(N // BN,),
        in_specs=[
            pl.BlockSpec((BN, H), lambda i: (i, 0)),
            pl.BlockSpec((H, H), lambda i: (0, 0)),
            pl.BlockSpec((1, H), lambda i: (0, 0)),
            pl.BlockSpec((H, H), lambda i: (0, 0)),
            pl.BlockSpec((1, H), lambda i: (0, 0)),
        ],
        out_specs=[
            pl.BlockSpec((BN, H), lambda i: (i, 0)),
            pl.BlockSpec((BN, H), lambda i: (i, 0)),
        ],
        out_shape=[
            jax.ShapeDtypeStruct((N, H), _f32),
            jax.ShapeDtypeStruct((N, H), _f32),
        ],
    )(x, params["in_W"], params["in_b"].reshape(1, H),
      p0["xp_W"], p0["xp_b"].reshape(1, H))


def _nku_body_proj(h_ref, hp_ref, a0_ref, a1_ref, d2_ref, g1a_ref, g1b_ref,
                   gb1_ref, g2_ref, gb2_ref, uw_ref, ub_ref, lng_ref, lnb_ref,
                   xw_ref, xb_ref, hn_ref, hpn_ref):
    hn = _nku_common(h_ref, hp_ref, a0_ref, a1_ref, d2_ref, g1a_ref, g1b_ref,
                     gb1_ref, g2_ref, gb2_ref, uw_ref, ub_ref, lng_ref,
                     lnb_ref)
    hn_ref[...] = hn
    hpn_ref[...] = _mm_t(hn, xw_ref[...]) + xb_ref[...]


def _nku_body_last(h_ref, hp_ref, a0_ref, a1_ref, d2_ref, g1a_ref, g1b_ref,
                   gb1_ref, g2_ref, gb2_ref, uw_ref, ub_ref, lng_ref, lnb_ref,
                   hn_ref):
    hn_ref[...] = _nku_common(h_ref, hp_ref, a0_ref, a1_ref, d2_ref, g1a_ref,
                              g1b_ref, gb1_ref, g2_ref, gb2_ref, uw_ref,
                              ub_ref, lng_ref, lnb_ref)


def _nku_common(h_ref, hp_ref, a0_ref, a1_ref, d2_ref, g1a_ref, g1b_ref,
                gb1_ref, g2_ref, gb2_ref, uw_ref, ub_ref, lng_ref, lnb_ref):
    ones_h1 = jnp.ones((H, 1), _f32)
    a = a0_ref[0] + a1_ref[0]
    dcol = lax.dot_general(d2_ref[...], jnp.ones((1, 2), _f32),
                           (((0,), (1,)), ((), ())),
                           preferred_element_type=_f32)
    inv_b = _mm_t(1.0 / (dcol + 1e-16), ones_h1)
    aggr = a * inv_b
    hpv = hp_ref[...]
    gi1 = (_mm_t(aggr, g1a_ref[...]) + _mm_t(hpv, g1b_ref[...])
           + gb1_ref[...])
    gate = jax.nn.sigmoid(_mm_t(jnp.maximum(gi1, 0.0), g2_ref[...])
                          + gb2_ref[...])
    gated = gate * aggr + (1.0 - gate) * hpv
    out = jnp.maximum(_mm_t(gated, uw_ref[...]) + ub_ref[...], 0.0)
    v = out + h_ref[...]
    ones_hh = jnp.full((H, H), 1.0 / H, _f32)
    mu_b = _mm_t(v, ones_hh)
    cen = v - mu_b
    var_b = _mm_t(cen * cen, ones_hh)
    return cen / jnp.sqrt(var_b + 1e-5) * lng_ref[...] + lnb_ref[...]


def _nku(h, hp, aggr2, den2, p, pnext):
    N = h.shape[0]
    last = pnext is None
    body = _nku_body_last if last else _nku_body_proj
    in_specs = [
        pl.BlockSpec((BN, H), lambda i: (i, 0)),
        pl.BlockSpec((BN, H), lambda i: (i, 0)),
        pl.BlockSpec((1, BN, H), lambda i: (0, i, 0)),
        pl.BlockSpec((1, BN, H), lambda i: (1, i, 0)),
        pl.BlockSpec((2, BN), lambda i: (0, i)),
        pl.BlockSpec((H, H), lambda i: (0, 0)),
        pl.BlockSpec((H, H), lambda i: (0, 0)),
        pl.BlockSpec((1, H), lambda i: (0, 0)),
        pl.BlockSpec((H, H), lambda i: (0, 0)),
        pl.BlockSpec((1, H), lambda i: (0, 0)),
        pl.BlockSpec((H, H), lambda i: (0, 0)),
        pl.BlockSpec((1, H), lambda i: (0, 0)),
        pl.BlockSpec((1, H), lambda i: (0, 0)),
        pl.BlockSpec((1, H), lambda i: (0, 0)),
    ]
    out_specs = [pl.BlockSpec((BN, H), lambda i: (i, 0))]
    out_shape = [jax.ShapeDtypeStruct((N, H), _f32)]
    args = [h, hp, aggr2, aggr2, den2,
            p["gate_W1"][:, :H], p["gate_W1"][:, H:],
            p["gate_b1"].reshape(1, H),
            jnp.broadcast_to(p["gate_W2"].reshape(1, H), (H, H)),
            jnp.broadcast_to(p["gate_b2"].reshape(1, 1), (1, H)),
            p["upd_W"], p["upd_b"].reshape(1, H),
            p["ln_g"].reshape(1, H), p["ln_b"].reshape(1, H)]
    if not last:
        in_specs += [pl.BlockSpec((H, H), lambda i: (0, 0)),
                     pl.BlockSpec((1, H), lambda i: (0, 0))]
        out_specs.append(pl.BlockSpec((BN, H), lambda i: (i, 0)))
        out_shape.append(jax.ShapeDtypeStruct((N, H), _f32))
        args += [pnext["xp_W"], pnext["xp_b"].reshape(1, H)]
    res = pl.pallas_call(
        body,
        grid=(N // BN,),
        in_specs=in_specs,
        out_specs=out_specs,
        out_shape=out_shape,
    )(*args)
    return (res[0], res[1]) if not last else (res[0], None)


def _readout_body(h_ref, st_ref, wv_ref, bv_ref, wo_ref, bo_ref, out_ref,
                  feats):
    for i in range(16):
        feats[pl.ds(i, 1), :] = h_ref[pl.ds(st_ref[i], 1), :]
    vv = _mm_t(feats[...], wv_ref[...]) + bv_ref[...]
    out_ref[...] = _mm_t(vv, wo_ref[...]) + bo_ref[...]


def _readout(h, starts, params):
    N = h.shape[0]
    return pl.pallas_call(
        _readout_body,
        in_specs=[
            pl.BlockSpec((N, H), lambda: (0, 0)),
            pl.BlockSpec(memory_space=pltpu.SMEM),
            pl.BlockSpec((H, H), lambda: (0, 0)),
            pl.BlockSpec((1, H), lambda: (0, 0)),
            pl.BlockSpec((H, H), lambda: (0, 0)),
            pl.BlockSpec((1, H), lambda: (0, 0)),
        ],
        out_specs=pl.BlockSpec((16, H), lambda: (0, 0)),
        out_shape=jax.ShapeDtypeStruct((16, H), _f32),
        scratch_shapes=[pltpu.VMEM((16, H), _f32)],
    )(h, starts, params["mha_Wv"], params["mha_bv"].reshape(1, H),
      params["mha_Wo"], params["mha_bo"].reshape(1, H))


def kernel(x, pos, params, edge_index, num_aa):
    N = x.shape[0]
    E = edge_index.shape[1]
    src = edge_index[0]
    dst = edge_index[1]
    pos4 = jnp.pad(pos, ((0, 0), (0, 1)))
    x = jnp.pad(x, ((0, NP - N), (0, 0)))

    prologue = _make_prologue(E, N)
    gather_rows = _make_gather_rows(E, N)
    segmax = _make_segmax(E)
    segden = _make_segden(E)
    segagg = _make_segagg(E)

    posf = pos4.reshape(N * 4)
    pef = jnp.concatenate([p["pos_emb"] for p in params["layers"]],
                          axis=1).reshape(512 * 48)
    vecf, pff = prologue(posf, src, dst, pef)
    vecT = vecf.reshape(4, E)
    pfT = pff.reshape(48, E)
    h, hp = _nk0(x, params)
    nl = len(params["layers"])
    for l, p in enumerate(params["layers"]):
        hsrc = gather_rows(hp, src)
        t, score = _edge_pass(hsrc, vecT, pfT, p, l)
        m2 = segmax(score, dst)
        den2, w = segden(m2, score, dst)
        aggr2 = segagg(w, dst, t)
        pnext = params["layers"][l + 1] if l + 1 < nl else None
        h, hp = _nku(h, hp, aggr2.reshape(2, NP, H), den2.reshape(2, NP),
                     p, pnext)

    starts = jnp.cumsum(num_aa) - num_aa
    return _readout(h, starts.astype(_i32), params)


# speedup vs baseline: 1.1956x; 1.1956x over previous
import functools

import jax
import jax.numpy as jnp
from jax import lax
from jax.experimental import pallas as pl
from jax.experimental.pallas import tpu as pltpu
from jax.experimental.pallas import tpu_sc as plsc

H = 128
RBF_K = 16
POS_EMB = 16
EB = 3200
BN = 2048
NP = 10240
NSL = NP // 16
NWORK = 32
RC = 80
SCH = 2000

_f32 = jnp.float32
_i32 = jnp.int32


def _mesh():
    return plsc.VectorSubcoreMesh(core_axis_name="c", subcore_axis_name="s")


def _wid():
    return lax.axis_index("s") * 2 + lax.axis_index("c")


def _mm_t(a, b):
    return lax.dot_general(a, b, (((1,), (1,)), ((), ())),
                           preferred_element_type=_f32)


def _make_prologue(E, N):
    TPE = E // NWORK
    NCH = TPE // RC

    @functools.partial(
        pl.kernel,
        out_type=[jax.ShapeDtypeStruct((4 * E,), _f32),
                  jax.ShapeDtypeStruct((48 * E,), _f32)],
        mesh=_mesh(),
        compiler_params=pltpu.CompilerParams(needs_layout_passes=False),
        scratch_types=[pltpu.VMEM((N * 4,), _f32),
                       pltpu.VMEM((512 * 48,), _f32),
                       pltpu.VMEM((RC,), _i32),
                       pltpu.VMEM((RC,), _i32),
                       pltpu.VMEM((RC,), _i32),
                       pltpu.VMEM((4 * RC,), _f32),
                       pltpu.VMEM((48 * RC,), _f32)],
    )
    def k(posf, srcr, dstr, pef, vec_o, pf_o,
          posv, pev, sbuf, dbuf, relb, vecb, pfb):
        base = _wid() * TPE
        iota = lax.iota(_i32, 16)
        pltpu.sync_copy(posf, posv)
        pltpu.sync_copy(pef, pev)

        def chunk(kk, _):
            off = base + kk * RC
            pltpu.sync_copy(srcr.at[pl.ds(off, RC)], sbuf)
            pltpu.sync_copy(dstr.at[pl.ds(off, RC)], dbuf)

            def vr(i, _):
                sl = pl.ds(i * 16, 16)
                s16 = sbuf[sl]
                d16 = dbuf[sl]
                s4 = s16 * 4
                d4 = d16 * 4
                for c in range(3):
                    a = plsc.load_gather(posv, [s4 + c])
                    b = plsc.load_gather(posv, [d4 + c])
                    vecb[pl.ds(c * RC + i * 16, 16)] = a - b
                vecb[pl.ds(3 * RC + i * 16, 16)] = jnp.zeros((16,), _f32)
                relb[sl] = jnp.clip(d16 - s16, 0, 511) * 48
                return 0

            lax.fori_loop(0, RC // 16, vr, 0)

            def edge(e, _):
                r16 = plsc.load_gather(relb, [jnp.full((16,), e, _i32)])
                for l in range(3):
                    pfb[pl.ds(e * 48 + l * 16, 16)] = plsc.load_gather(
                        pev, [r16 + l * 16 + iota])
                return 0

            lax.fori_loop(0, RC, edge, 0)
            for c in range(4):
                pltpu.sync_copy(vecb.at[pl.ds(c * RC, RC)],
                                vec_o.at[pl.ds(c * E + off, RC)])
            pltpu.sync_copy(pfb, pf_o.at[pl.ds(off * 48, RC * 48)])
            return 0

        lax.fori_loop(0, NCH, chunk, 0)

    return k


def _make_gather_rows(E, N):
    TPE = E // NWORK
    NCH = TPE // RC

    NBUF = 5

    @functools.partial(
        pl.kernel,
        out_type=jax.ShapeDtypeStruct((E, H), _f32),
        mesh=_mesh(),
        compiler_params=pltpu.CompilerParams(needs_layout_passes=False),
        scratch_types=[pltpu.VMEM((TPE,), _i32)]
        + [pltpu.VMEM((RC, H), _f32) for _ in range(NBUF)]
        + [pltpu.SemaphoreType.DMA for _ in range(NBUF)],
    )
    def k(hp, srcr, out, ibuf, *bufsem):
        rbs = bufsem[:NBUF]
        sems = bufsem[NBUF:]
        base = _wid() * TPE
        pltpu.sync_copy(srcr.at[pl.ds(base, TPE)], ibuf)

        def group(kk, _):
            g0 = kk * (NBUF * RC)
            cps = [
                pltpu.async_copy(
                    hp.at[ibuf.at[pl.ds(g0 + j * RC, RC)]], rbs[j], sems[j])
                for j in range(NBUF)
            ]
            for j in range(NBUF):
                cps[j].wait()
                pltpu.sync_copy(rbs[j],
                                out.at[pl.ds(base + g0 + j * RC, RC)])
            return 0

        lax.fori_loop(0, NCH // NBUF, group, 0)

    return k


def _make_segmax(E):
    TPE = E // NWORK
    NCH = TPE // SCH

    @functools.partial(
        pl.kernel,
        out_type=jax.ShapeDtypeStruct((2 * NP,), _f32),
        mesh=_mesh(),
        compiler_params=pltpu.CompilerParams(needs_layout_passes=False),
        scratch_types=[pltpu.VMEM((NP,), _f32),
                       pltpu.VMEM((SCH,), _i32),
                       pltpu.VMEM((SCH,), _f32),
                       pltpu.VMEM((NSL,), _f32),
                       pltpu.VMEM((NSL,), _f32),
                       pltpu.VMEM_SHARED((16 * NP,), _f32)],
    )
    def k(score, dstr, m2, mloc, dbuf, sbuf, acc, tmp, shm):
        cid = lax.axis_index("c")
        sid = lax.axis_index("s")
        base = _wid() * TPE
        iota = lax.iota(_i32, 16)

        def z(i, _):
            mloc[pl.ds(i * 16, 16)] = jnp.full((16,), -1e30, _f32)
            return 0

        lax.fori_loop(0, NP // 16, z, 0)

        def chunk(kk, _):
            off = base + kk * SCH
            pltpu.sync_copy(dstr.at[pl.ds(off, SCH)], dbuf)
            pltpu.sync_copy(score.at[pl.ds(off, SCH)], sbuf)

            def vr(i, _):
                d0 = dbuf[pl.ds(i * 16, 16)]
                s0 = sbuf[pl.ds(i * 16, 16)]
                d, s = plsc.sort_key_val(d0, s0)
                run = s
                for ksh in (1, 2, 4, 8):
                    idx = jnp.maximum(iota - ksh, 0)
                    dsh = d.at[idx].get(mode="promise_in_bounds")
                    rsh = run.at[idx].get(mode="promise_in_bounds")
                    ok = (iota >= ksh) & (dsh == d)
                    run = jnp.where(ok, jnp.maximum(run, rsh), run)
                nxt = jnp.minimum(iota + 1, 15)
                dn = d.at[nxt].get(mode="promise_in_bounds")
                last = (dn != d) | (iota == 15)
                cur = plsc.load_gather(mloc, [d])
                plsc.store_scatter(mloc, [d], jnp.maximum(cur, run),
                                   mask=last)
                return 0

            lax.fori_loop(0, SCH // 16, vr, 0)
            return 0

        lax.fori_loop(0, NCH, chunk, 0)

        pltpu.sync_copy(mloc, shm.at[pl.ds(sid * NP, NP)])
        plsc.subcore_barrier()
        s0 = sid * NSL
        pltpu.sync_copy(shm.at[pl.ds(s0, NSL)], acc)
        for r in range(1, 16):
            pltpu.sync_copy(shm.at[pl.ds(r * NP + s0, NSL)], tmp)

            def mx(i, _):
                acc[pl.ds(i * 16, 16)] = jnp.maximum(
                    acc[pl.ds(i * 16, 16)], tmp[pl.ds(i * 16, 16)])
                return 0

            lax.fori_loop(0, NSL // 16, mx, 0)
        pltpu.sync_copy(acc, m2.at[pl.ds(cid * NP + s0, NSL)])

    return k


def _make_segden(E):
    TPE = E // NWORK
    NCH = TPE // SCH

    @functools.partial(
        pl.kernel,
        out_type=[jax.ShapeDtypeStruct((2 * NP,), _f32),
                  jax.ShapeDtypeStruct((E,), _f32)],
        mesh=_mesh(),
        compiler_params=pltpu.CompilerParams(needs_layout_passes=False),
        scratch_types=[pltpu.VMEM((NP,), _f32),
                       pltpu.VMEM((NP,), _f32),
                       pltpu.VMEM((NP,), _f32),
                       pltpu.VMEM((SCH,), _i32),
                       pltpu.VMEM((SCH,), _f32),
                       pltpu.VMEM((NSL,), _f32),
                       pltpu.VMEM((NSL,), _f32),
                       pltpu.VMEM_SHARED((16 * NP,), _f32)],
    )
    def k(m2, score, dstr, den2, w_o, ma, mb, den, dbuf, sbuf, acc, tmp,
          shm):
        cid = lax.axis_index("c")
        sid = lax.axis_index("s")
        base = _wid() * TPE
        pltpu.sync_copy(m2.at[pl.ds(0, NP)], ma)
        pltpu.sync_copy(m2.at[pl.ds(NP, NP)], mb)

        def mmax(i, _):
            ma[pl.ds(i * 16, 16)] = jnp.maximum(ma[pl.ds(i * 16, 16)],
                                                mb[pl.ds(i * 16, 16)])
            return 0

        lax.fori_loop(0, NP // 16, mmax, 0)

        def z(i, _):
            den[pl.ds(i * 16, 16)] = jnp.zeros((16,), _f32)
            return 0

        lax.fori_loop(0, NP // 16, z, 0)

        def chunk(kk, _):
            off = base + kk * SCH
            pltpu.sync_copy(dstr.at[pl.ds(off, SCH)], dbuf)
            pltpu.sync_copy(score.at[pl.ds(off, SCH)], sbuf)

            def vr(i, _):
                d = dbuf[pl.ds(i * 16, 16)]
                s = sbuf[pl.ds(i * 16, 16)]
                mv = plsc.load_gather(ma, [d])
                ex = jnp.exp(s - mv)
                sbuf[pl.ds(i * 16, 16)] = ex
                plsc.addupdate_scatter(den, [d], ex)
                return 0

            lax.fori_loop(0, SCH // 16, vr, 0)
            pltpu.sync_copy(sbuf, w_o.at[pl.ds(off, SCH)])
            return 0

        lax.fori_loop(0, NCH, chunk, 0)

        pltpu.sync_copy(den, shm.at[pl.ds(sid * NP, NP)])
        plsc.subcore_barrier()
        s0 = sid * NSL
        pltpu.sync_copy(shm.at[pl.ds(s0, NSL)], acc)
        for r in range(1, 16):
            pltpu.sync_copy(shm.at[pl.ds(r * NP + s0, NSL)], tmp)

            def ad(i, _):
                acc[pl.ds(i * 16, 16)] = (acc[pl.ds(i * 16, 16)]
                                          + tmp[pl.ds(i * 16, 16)])
                return 0

            lax.fori_loop(0, NSL // 16, ad, 0)
        pltpu.sync_copy(acc, den2.at[pl.ds(cid * NP + s0, NSL)])

    return k


def _make_segagg(E):
    TPE = E // NWORK
    NCH = TPE // RC
    ZR = 64

    @functools.partial(
        pl.kernel,
        out_type=jax.ShapeDtypeStruct((2 * NP, H), _f32),
        mesh=_mesh(),
        compiler_params=pltpu.CompilerParams(needs_layout_passes=False),
        scratch_types=[pltpu.VMEM((RC,), _i32),
                       pltpu.VMEM((RC,), _f32),
                       pltpu.VMEM((RC, H), _f32),
                       pltpu.VMEM((ZR, H), _f32),
                       pltpu.VMEM_SHARED((NP, H), _f32)],
    )
    def k(w, dstr, t, aggr2, idxb, wb, tb, zb, ash):
        cid = lax.axis_index("c")
        sid = lax.axis_index("s")
        base = _wid() * TPE

        def zz(i, _):
            for j in range(H // 16):
                zb[i, pl.ds(j * 16, 16)] = jnp.zeros((16,), _f32)
            return 0

        lax.fori_loop(0, ZR, zz, 0)
        s0 = sid * NSL
        for j in range(NSL // ZR):
            pltpu.sync_copy(zb, ash.at[pl.ds(s0 + j * ZR, ZR)])
        plsc.subcore_barrier()

        def chunk(kk, _):
            off = base + kk * RC
            pltpu.sync_copy(dstr.at[pl.ds(off, RC)], idxb)
            pltpu.sync_copy(w.at[pl.ds(off, RC)], wb)
            pltpu.sync_copy(t.at[pl.ds(off, RC)], tb)

            def row(e, _):
                ws = plsc.load_gather(wb, [jnp.full((16,), e, _i32)])
                for j in range(H // 16):
                    tb[e, pl.ds(j * 16, 16)] = tb[e, pl.ds(j * 16, 16)] * ws
                return 0

            lax.fori_loop(0, RC, row, 0)
            pltpu.sync_copy(tb, ash.at[idxb], add=True)
            return 0

        lax.fori_loop(0, NCH, chunk, 0)
        plsc.subcore_barrier()
        pltpu.sync_copy(ash.at[pl.ds(s0, NSL)],
                        aggr2.at[pl.ds(cid * NP + s0, NSL)])

    return k


def _edge_body(hs_ref, vec_ref, pf_ref, rbfw_ref, w1r_ref, w1p_ref,
               w1d_ref, b1_ref, w2_ref, b2_ref, t_ref, score_ref):
    v4 = vec_ref[...]
    d2r = jnp.sum(v4 * v4, axis=0, keepdims=True)
    dcol = lax.dot_general(d2r, jnp.ones((1, 1), _f32),
                           (((0,), (1,)), ((), ())),
                           preferred_element_type=_f32)
    dist_col = jnp.sqrt(dcol)
    rbf = jnp.exp(-jnp.square(_mm_t(dist_col, rbfw_ref[...])))
    invdr = 1.0 / jnp.clip(jnp.sqrt(d2r), 1e-12, None)
    dir4 = v4 * invdr
    f_dir = lax.dot_general(dir4, w1d_ref[...],
                            (((0,), (1,)), ((), ())),
                            preferred_element_type=_f32)
    f_pos = _mm_t(pf_ref[...], w1p_ref[...])
    f1 = _mm_t(rbf, w1r_ref[...]) + f_pos + f_dir + b1_ref[...]
    f = _mm_t(jnp.maximum(f1, 0.0), w2_ref[...]) + b2_ref[...]
    hs = hs_ref[...]
    score_ref[...] = jnp.sum(hs * f, axis=1, keepdims=True)
    t_ref[...] = hs + f


def _edge_pass(hsrc, vecT, pfA, p, lidx):
    E = hsrc.shape[0]
    rbfw = p["rbf_W"].reshape(RBF_K, 1)
    w1 = p["frame_W1"]
    w1r = w1[:, :RBF_K]
    w1p = jnp.pad(w1[:, RBF_K:RBF_K + POS_EMB],
                  ((0, 0), (16 * lidx, 48 - 16 * lidx - 16)))
    w1d = jnp.pad(w1[:, RBF_K + POS_EMB:], ((0, 0), (0, 1)))
    b1 = p["frame_b1"].reshape(1, H)
    w2 = p["frame_W2"]
    b2 = p["frame_b2"].reshape(1, H)
    grid = E // EB
    t, score = pl.pallas_call(
        _edge_body,
        grid=(grid,),
        in_specs=[
            pl.BlockSpec((EB, H), lambda i: (i, 0)),
            pl.BlockSpec((4, EB), lambda i: (0, i)),
            pl.BlockSpec((EB, 48), lambda i: (i, 0)),
            pl.BlockSpec((RBF_K, 1), lambda i: (0, 0)),
            pl.BlockSpec((H, RBF_K), lambda i: (0, 0)),
            pl.BlockSpec((H, 48), lambda i: (0, 0)),
            pl.BlockSpec((H, 4), lambda i: (0, 0)),
            pl.BlockSpec((1, H), lambda i: (0, 0)),
            pl.BlockSpec((H, H), lambda i: (0, 0)),
            pl.BlockSpec((1, H), lambda i: (0, 0)),
        ],
        out_specs=[
            pl.BlockSpec((EB, H), lambda i: (i, 0)),
            pl.BlockSpec((EB, 1), lambda i: (i, 0)),
        ],
        out_shape=[
            jax.ShapeDtypeStruct((E, H), _f32),
            jax.ShapeDtypeStruct((E, 1), _f32),
        ],
    )(hsrc, vecT, pfA, rbfw, w1r, w1p, w1d, b1, w2, b2)
    return t, score.reshape(E)


def _nk0_body(x_ref, w_ref, b_ref, xw_ref, xb_ref, h_ref, hp_ref):
    hv = _mm_t(x_ref[...], w_ref[...]) + b_ref[...]
    h_ref[...] = hv
    hp_ref[...] = _mm_t(hv, xw_ref[...]) + xb_ref[...]


def _nk0(x, params):
    N = x.shape[0]
    p0 = params["layers"][0]
    return pl.pallas_call(
        _nk0_body,
        grid=(N // BN,),
        in_specs=[
            pl.BlockSpec((BN, H), lambda i: (i, 0)),
            pl.BlockSpec((H, H), lambda i: (0, 0)),
            pl.BlockSpec((1, H), lambda i: (0, 0)),
            pl.BlockSpec((H, H), lambda i: (0, 0)),
            pl.BlockSpec((1, H), lambda i: (0, 0)),
        ],
        out_specs=[
            pl.BlockSpec((BN, H), lambda i: (i, 0)),
            pl.BlockSpec((BN, H), lambda i: (i, 0)),
        ],
        out_shape=[
            jax.ShapeDtypeStruct((N, H), _f32),
            jax.ShapeDtypeStruct((N, H), _f32),
        ],
    )(x, params["in_W"], params["in_b"].reshape(1, H),
      p0["xp_W"], p0["xp_b"].reshape(1, H))


def _nku_body_proj(h_ref, hp_ref, a0_ref, a1_ref, d2_ref, g1a_ref, g1b_ref,
                   gb1_ref, g2_ref, gb2_ref, uw_ref, ub_ref, lng_ref, lnb_ref,
                   xw_ref, xb_ref, hn_ref, hpn_ref):
    hn = _nku_common(h_ref, hp_ref, a0_ref, a1_ref, d2_ref, g1a_ref, g1b_ref,
                     gb1_ref, g2_ref, gb2_ref, uw_ref, ub_ref, lng_ref,
                     lnb_ref)
    hn_ref[...] = hn
    hpn_ref[...] = _mm_t(hn, xw_ref[...]) + xb_ref[...]


def _nku_body_last(h_ref, hp_ref, a0_ref, a1_ref, d2_ref, g1a_ref, g1b_ref,
                   gb1_ref, g2_ref, gb2_ref, uw_ref, ub_ref, lng_ref, lnb_ref,
                   hn_ref):
    hn_ref[...] = _nku_common(h_ref, hp_ref, a0_ref, a1_ref, d2_ref, g1a_ref,
                              g1b_ref, gb1_ref, g2_ref, gb2_ref, uw_ref,
                              ub_ref, lng_ref, lnb_ref)


def _nku_common(h_ref, hp_ref, a0_ref, a1_ref, d2_ref, g1a_ref, g1b_ref,
                gb1_ref, g2_ref, gb2_ref, uw_ref, ub_ref, lng_ref, lnb_ref):
    ones_h1 = jnp.ones((H, 1), _f32)
    a = a0_ref[0] + a1_ref[0]
    dcol = lax.dot_general(d2_ref[...], jnp.ones((1, 2), _f32),
                           (((0,), (1,)), ((), ())),
                           preferred_element_type=_f32)
    inv_b = _mm_t(1.0 / (dcol + 1e-16), ones_h1)
    aggr = a * inv_b
    hpv = hp_ref[...]
    gi1 = (_mm_t(aggr, g1a_ref[...]) + _mm_t(hpv, g1b_ref[...])
           + gb1_ref[...])
    gate = jax.nn.sigmoid(_mm_t(jnp.maximum(gi1, 0.0), g2_ref[...])
                          + gb2_ref[...])
    gated = gate * aggr + (1.0 - gate) * hpv
    out = jnp.maximum(_mm_t(gated, uw_ref[...]) + ub_ref[...], 0.0)
    v = out + h_ref[...]
    ones_hh = jnp.full((H, H), 1.0 / H, _f32)
    mu_b = _mm_t(v, ones_hh)
    cen = v - mu_b
    var_b = _mm_t(cen * cen, ones_hh)
    return cen / jnp.sqrt(var_b + 1e-5) * lng_ref[...] + lnb_ref[...]


def _nku(h, hp, aggr2, den2, p, pnext):
    N = h.shape[0]
    last = pnext is None
    body = _nku_body_last if last else _nku_body_proj
    in_specs = [
        pl.BlockSpec((BN, H), lambda i: (i, 0)),
        pl.BlockSpec((BN, H), lambda i: (i, 0)),
        pl.BlockSpec((1, BN, H), lambda i: (0, i, 0)),
        pl.BlockSpec((1, BN, H), lambda i: (1, i, 0)),
        pl.BlockSpec((2, BN), lambda i: (0, i)),
        pl.BlockSpec((H, H), lambda i: (0, 0)),
        pl.BlockSpec((H, H), lambda i: (0, 0)),
        pl.BlockSpec((1, H), lambda i: (0, 0)),
        pl.BlockSpec((H, H), lambda i: (0, 0)),
        pl.BlockSpec((1, H), lambda i: (0, 0)),
        pl.BlockSpec((H, H), lambda i: (0, 0)),
        pl.BlockSpec((1, H), lambda i: (0, 0)),
        pl.BlockSpec((1, H), lambda i: (0, 0)),
        pl.BlockSpec((1, H), lambda i: (0, 0)),
    ]
    out_specs = [pl.BlockSpec((BN, H), lambda i: (i, 0))]
    out_shape = [jax.ShapeDtypeStruct((N, H), _f32)]
    args = [h, hp, aggr2, aggr2, den2,
            p["gate_W1"][:, :H], p["gate_W1"][:, H:],
            p["gate_b1"].reshape(1, H),
            jnp.broadcast_to(p["gate_W2"].reshape(1, H), (H, H)),
            jnp.broadcast_to(p["gate_b2"].reshape(1, 1), (1, H)),
            p["upd_W"], p["upd_b"].reshape(1, H),
            p["ln_g"].reshape(1, H), p["ln_b"].reshape(1, H)]
    if not last:
        in_specs += [pl.BlockSpec((H, H), lambda i: (0, 0)),
                     pl.BlockSpec((1, H), lambda i: (0, 0))]
        out_specs.append(pl.BlockSpec((BN, H), lambda i: (i, 0)))
        out_shape.append(jax.ShapeDtypeStruct((N, H), _f32))
        args += [pnext["xp_W"], pnext["xp_b"].reshape(1, H)]
    res = pl.pallas_call(
        body,
        grid=(N // BN,),
        in_specs=in_specs,
        out_specs=out_specs,
        out_shape=out_shape,
    )(*args)
    return (res[0], res[1]) if not last else (res[0], None)


def _readout_body(h_ref, st_ref, wv_ref, bv_ref, wo_ref, bo_ref, out_ref,
                  feats):
    for i in range(16):
        feats[pl.ds(i, 1), :] = h_ref[pl.ds(st_ref[i], 1), :]
    vv = _mm_t(feats[...], wv_ref[...]) + bv_ref[...]
    out_ref[...] = _mm_t(vv, wo_ref[...]) + bo_ref[...]


def _readout(h, starts, params):
    N = h.shape[0]
    return pl.pallas_call(
        _readout_body,
        in_specs=[
            pl.BlockSpec((N, H), lambda: (0, 0)),
            pl.BlockSpec(memory_space=pltpu.SMEM),
            pl.BlockSpec((H, H), lambda: (0, 0)),
            pl.BlockSpec((1, H), lambda: (0, 0)),
            pl.BlockSpec((H, H), lambda: (0, 0)),
            pl.BlockSpec((1, H), lambda: (0, 0)),
        ],
        out_specs=pl.BlockSpec((16, H), lambda: (0, 0)),
        out_shape=jax.ShapeDtypeStruct((16, H), _f32),
        scratch_shapes=[pltpu.VMEM((16, H), _f32)],
    )(h, starts, params["mha_Wv"], params["mha_bv"].reshape(1, H),
      params["mha_Wo"], params["mha_bo"].reshape(1, H))


def kernel(x, pos, params, edge_index, num_aa):
    N = x.shape[0]
    E = edge_index.shape[1]
    src = edge_index[0]
    dst = edge_index[1]
    pos4 = jnp.pad(pos, ((0, 0), (0, 1)))
    x = jnp.pad(x, ((0, NP - N), (0, 0)))

    prologue = _make_prologue(E, N)
    gather_rows = _make_gather_rows(E, N)
    segmax = _make_segmax(E)
    segden = _make_segden(E)
    segagg = _make_segagg(E)

    posf = pos4.reshape(N * 4)
    pef = jnp.concatenate([p["pos_emb"] for p in params["layers"]],
                          axis=1).reshape(512 * 48)
    vecf, pff = prologue(posf, src, dst, pef)
    vecT = vecf.reshape(4, E)
    pfA = pff.reshape(E, 48)
    h, hp = _nk0(x, params)
    nl = len(params["layers"])
    for l, p in enumerate(params["layers"]):
        hsrc = gather_rows(hp, src)
        t, score = _edge_pass(hsrc, vecT, pfA, p, l)
        m2 = segmax(score, dst)
        den2, w = segden(m2, score, dst)
        aggr2 = segagg(w, dst, t)
        pnext = params["layers"][l + 1] if l + 1 < nl else None
        h, hp = _nku(h, hp, aggr2.reshape(2, NP, H), den2.reshape(2, NP),
                     p, pnext)

    starts = jnp.cumsum(num_aa) - num_aa
    return _readout(h, starts.astype(_i32), params)


# speedup vs baseline: 1.3428x; 1.1231x over previous
import functools

import jax
import jax.numpy as jnp
from jax import lax
from jax.experimental import pallas as pl
from jax.experimental.pallas import tpu as pltpu
from jax.experimental.pallas import tpu_sc as plsc

H = 128
RBF_K = 16
POS_EMB = 16
EB = 3200
BN = 2048
NP = 10240
NSL = NP // 16
NWORK = 32
RC = 80
SCH = 2000

_f32 = jnp.float32
_i32 = jnp.int32


def _mesh():
    return plsc.VectorSubcoreMesh(core_axis_name="c", subcore_axis_name="s")


def _wid():
    return lax.axis_index("s") * 2 + lax.axis_index("c")


def _mm_t(a, b):
    return lax.dot_general(a, b, (((1,), (1,)), ((), ())),
                           preferred_element_type=_f32)


def _make_prologue(E, N):
    TPE = E // NWORK
    NCH = TPE // RC

    @functools.partial(
        pl.kernel,
        out_type=[jax.ShapeDtypeStruct((4 * E,), _f32),
                  jax.ShapeDtypeStruct((48 * E,), _f32)],
        mesh=_mesh(),
        compiler_params=pltpu.CompilerParams(needs_layout_passes=False),
        scratch_types=[pltpu.VMEM((N * 4,), _f32),
                       pltpu.VMEM((512 * 48,), _f32),
                       pltpu.VMEM((RC,), _i32),
                       pltpu.VMEM((RC,), _i32),
                       pltpu.VMEM((RC,), _i32),
                       pltpu.VMEM((4 * RC,), _f32),
                       pltpu.VMEM((48 * RC,), _f32)],
    )
    def k(posf, srcr, dstr, pef, vec_o, pf_o,
          posv, pev, sbuf, dbuf, relb, vecb, pfb):
        base = _wid() * TPE
        iota = lax.iota(_i32, 16)
        pltpu.sync_copy(posf, posv)
        pltpu.sync_copy(pef, pev)

        def chunk(kk, _):
            off = base + kk * RC
            pltpu.sync_copy(srcr.at[pl.ds(off, RC)], sbuf)
            pltpu.sync_copy(dstr.at[pl.ds(off, RC)], dbuf)

            def vr(i, _):
                sl = pl.ds(i * 16, 16)
                s16 = sbuf[sl]
                d16 = dbuf[sl]
                s4 = s16 * 4
                d4 = d16 * 4
                for c in range(3):
                    a = plsc.load_gather(posv, [s4 + c])
                    b = plsc.load_gather(posv, [d4 + c])
                    vecb[pl.ds(c * RC + i * 16, 16)] = a - b
                vecb[pl.ds(3 * RC + i * 16, 16)] = jnp.zeros((16,), _f32)
                relb[sl] = jnp.clip(d16 - s16, 0, 511) * 48
                return 0

            lax.fori_loop(0, RC // 16, vr, 0)

            def edge(e, _):
                r16 = plsc.load_gather(relb, [jnp.full((16,), e, _i32)])
                for l in range(3):
                    pfb[pl.ds(e * 48 + l * 16, 16)] = plsc.load_gather(
                        pev, [r16 + l * 16 + iota])
                return 0

            lax.fori_loop(0, RC, edge, 0)
            for c in range(4):
                pltpu.sync_copy(vecb.at[pl.ds(c * RC, RC)],
                                vec_o.at[pl.ds(c * E + off, RC)])
            pltpu.sync_copy(pfb, pf_o.at[pl.ds(off * 48, RC * 48)])
            return 0

        lax.fori_loop(0, NCH, chunk, 0)

    return k


def _make_gather_rows(E, N):
    TPE = E // NWORK
    NCH = TPE // RC

    NBUF = 5

    @functools.partial(
        pl.kernel,
        out_type=jax.ShapeDtypeStruct((E, H), _f32),
        mesh=_mesh(),
        compiler_params=pltpu.CompilerParams(needs_layout_passes=False),
        scratch_types=[pltpu.VMEM((TPE,), _i32)]
        + [pltpu.VMEM((RC, H), _f32) for _ in range(NBUF)]
        + [pltpu.SemaphoreType.DMA for _ in range(NBUF)],
    )
    def k(hp, srcr, out, ibuf, *bufsem):
        rbs = bufsem[:NBUF]
        sems = bufsem[NBUF:]
        base = _wid() * TPE
        pltpu.sync_copy(srcr.at[pl.ds(base, TPE)], ibuf)

        def group(kk, _):
            g0 = kk * (NBUF * RC)
            cps = [
                pltpu.async_copy(
                    hp.at[ibuf.at[pl.ds(g0 + j * RC, RC)]], rbs[j], sems[j])
                for j in range(NBUF)
            ]
            for j in range(NBUF):
                cps[j].wait()
                pltpu.sync_copy(rbs[j],
                                out.at[pl.ds(base + g0 + j * RC, RC)])
            return 0

        lax.fori_loop(0, NCH // NBUF, group, 0)

    return k


def _make_segmax(E):
    TPE = E // NWORK
    NCH = TPE // SCH

    @functools.partial(
        pl.kernel,
        out_type=jax.ShapeDtypeStruct((2 * NP,), _f32),
        mesh=_mesh(),
        compiler_params=pltpu.CompilerParams(needs_layout_passes=False),
        scratch_types=[pltpu.VMEM((NP,), _f32),
                       pltpu.VMEM((SCH,), _i32),
                       pltpu.VMEM((SCH,), _f32),
                       pltpu.VMEM((NSL,), _f32),
                       pltpu.VMEM((NSL,), _f32),
                       pltpu.VMEM_SHARED((16 * NP,), _f32)],
    )
    def k(score, dstr, m2, mloc, dbuf, sbuf, acc, tmp, shm):
        cid = lax.axis_index("c")
        sid = lax.axis_index("s")
        base = _wid() * TPE
        iota = lax.iota(_i32, 16)

        def z(i, _):
            mloc[pl.ds(i * 16, 16)] = jnp.full((16,), -1e30, _f32)
            return 0

        lax.fori_loop(0, NP // 16, z, 0)

        def chunk(kk, _):
            off = base + kk * SCH
            pltpu.sync_copy(dstr.at[pl.ds(off, SCH)], dbuf)
            pltpu.sync_copy(score.at[pl.ds(off, SCH)], sbuf)

            def vr(i, _):
                d0 = dbuf[pl.ds(i * 16, 16)]
                s0 = sbuf[pl.ds(i * 16, 16)]
                d, s = plsc.sort_key_val(d0, s0)
                run = s
                for ksh in (1, 2, 4, 8):
                    idx = jnp.maximum(iota - ksh, 0)
                    dsh = d.at[idx].get(mode="promise_in_bounds")
                    rsh = run.at[idx].get(mode="promise_in_bounds")
                    ok = (iota >= ksh) & (dsh == d)
                    run = jnp.where(ok, jnp.maximum(run, rsh), run)
                nxt = jnp.minimum(iota + 1, 15)
                dn = d.at[nxt].get(mode="promise_in_bounds")
                last = (dn != d) | (iota == 15)
                cur = plsc.load_gather(mloc, [d])
                plsc.store_scatter(mloc, [d], jnp.maximum(cur, run),
                                   mask=last)
                return 0

            lax.fori_loop(0, SCH // 16, vr, 0)
            return 0

        lax.fori_loop(0, NCH, chunk, 0)

        pltpu.sync_copy(mloc, shm.at[pl.ds(sid * NP, NP)])
        plsc.subcore_barrier()
        s0 = sid * NSL
        pltpu.sync_copy(shm.at[pl.ds(s0, NSL)], acc)
        for r in range(1, 16):
            pltpu.sync_copy(shm.at[pl.ds(r * NP + s0, NSL)], tmp)

            def mx(i, _):
                acc[pl.ds(i * 16, 16)] = jnp.maximum(
                    acc[pl.ds(i * 16, 16)], tmp[pl.ds(i * 16, 16)])
                return 0

            lax.fori_loop(0, NSL // 16, mx, 0)
        pltpu.sync_copy(acc, m2.at[pl.ds(cid * NP + s0, NSL)])

    return k


def _make_segdenagg(E):
    TPE = E // NWORK
    NCH = TPE // RC
    ZR = 64

    @functools.partial(
        pl.kernel,
        out_type=[jax.ShapeDtypeStruct((NWORK * NP,), _f32),
                  jax.ShapeDtypeStruct((2 * NP, H), _f32)],
        mesh=_mesh(),
        compiler_params=pltpu.CompilerParams(needs_layout_passes=False),
        scratch_types=[pltpu.VMEM((NP,), _f32),
                       pltpu.VMEM((NP,), _f32),
                       pltpu.VMEM((RC, H), _f32),
                       pltpu.VMEM((RC, H), _f32),
                       pltpu.VMEM((RC,), _i32),
                       pltpu.VMEM((RC,), _i32),
                       pltpu.VMEM((RC,), _f32),
                       pltpu.VMEM((RC,), _f32),
                       pltpu.VMEM_SHARED((NP, H), _f32),
                       pltpu.SemaphoreType.DMA,
                       pltpu.SemaphoreType.DMA],
    )
    def k(m2, score, dstr, t, den2, aggr2, ma, den, tb0, tb1, ib0, ib1,
          sb0, sb1, ash, sm0, sm1):
        cid = lax.axis_index("c")
        sid = lax.axis_index("s")
        base = _wid() * TPE
        s0 = sid * NSL
        pltpu.sync_copy(m2.at[pl.ds(0, NP)], ma)
        pltpu.sync_copy(m2.at[pl.ds(NP, NP)], den)

        def mx(i, _):
            ma[pl.ds(i * 16, 16)] = jnp.maximum(ma[pl.ds(i * 16, 16)],
                                                den[pl.ds(i * 16, 16)])
            return 0

        lax.fori_loop(0, NP // 16, mx, 0)

        def z(i, _):
            den[pl.ds(i * 16, 16)] = jnp.zeros((16,), _f32)
            return 0

        lax.fori_loop(0, NP // 16, z, 0)

        def zz(i, _):
            for j in range(H // 16):
                tb0[i, pl.ds(j * 16, 16)] = jnp.zeros((16,), _f32)
            return 0

        lax.fori_loop(0, RC, zz, 0)
        for j in range(NSL // RC):
            pltpu.sync_copy(tb0, ash.at[pl.ds(s0 + j * RC, RC)])
        plsc.subcore_barrier()

        def half(o0, ib, sb, tb, cp):
            def ex(i, _):
                d = ib[pl.ds(i * 16, 16)]
                mv = plsc.load_gather(ma, [d])
                e16 = jnp.exp(sb[pl.ds(i * 16, 16)] - mv)
                sb[pl.ds(i * 16, 16)] = e16
                plsc.addupdate_scatter(den, [d], e16)
                return 0

            lax.fori_loop(0, RC // 16, ex, 0)
            cp.wait()

            def row(e, _):
                ws = plsc.load_gather(sb, [jnp.full((16,), e, _i32)])
                for j in range(H // 16):
                    tb[e, pl.ds(j * 16, 16)] = tb[e, pl.ds(j * 16, 16)] * ws
                return 0

            lax.fori_loop(0, RC, row, 0)
            pltpu.sync_copy(tb, ash.at[ib], add=True)

        def grp(kk, _):
            o0 = base + kk * 2 * RC
            c0 = pltpu.async_copy(t.at[pl.ds(o0, RC)], tb0, sm0)
            c1 = pltpu.async_copy(t.at[pl.ds(o0 + RC, RC)], tb1, sm1)
            pltpu.sync_copy(dstr.at[pl.ds(o0, RC)], ib0)
            pltpu.sync_copy(score.at[pl.ds(o0, RC)], sb0)
            pltpu.sync_copy(dstr.at[pl.ds(o0 + RC, RC)], ib1)
            pltpu.sync_copy(score.at[pl.ds(o0 + RC, RC)], sb1)
            half(o0, ib0, sb0, tb0, c0)
            half(o0 + RC, ib1, sb1, tb1, c1)
            return 0

        lax.fori_loop(0, NCH // 2, grp, 0)

        if NCH % 2:
            oT = base + (NCH - 1) * RC
            cT = pltpu.async_copy(t.at[pl.ds(oT, RC)], tb0, sm0)
            pltpu.sync_copy(dstr.at[pl.ds(oT, RC)], ib0)
            pltpu.sync_copy(score.at[pl.ds(oT, RC)], sb0)
            half(oT, ib0, sb0, tb0, cT)

        pltpu.sync_copy(den, den2.at[pl.ds(_wid() * NP, NP)])
        plsc.subcore_barrier()
        pltpu.sync_copy(ash.at[pl.ds(s0, NSL)],
                        aggr2.at[pl.ds(cid * NP + s0, NSL)])

    return k


def _edge_body(hs_ref, vec_ref, pf_ref, rbfw_ref, w1r_ref, w1p_ref,
               w1d_ref, b1_ref, w2_ref, b2_ref, t_ref, score_ref):
    v4 = vec_ref[...]
    d2r = jnp.sum(v4 * v4, axis=0, keepdims=True)
    dcol = lax.dot_general(d2r, jnp.ones((1, 1), _f32),
                           (((0,), (1,)), ((), ())),
                           preferred_element_type=_f32)
    dist_col = jnp.sqrt(dcol)
    rbf = jnp.exp(-jnp.square(_mm_t(dist_col, rbfw_ref[...])))
    invdr = 1.0 / jnp.clip(jnp.sqrt(d2r), 1e-12, None)
    dir4 = v4 * invdr
    f_dir = lax.dot_general(dir4, w1d_ref[...],
                            (((0,), (1,)), ((), ())),
                            preferred_element_type=_f32)
    f_pos = _mm_t(pf_ref[...], w1p_ref[...])
    f1 = _mm_t(rbf, w1r_ref[...]) + f_pos + f_dir + b1_ref[...]
    f = _mm_t(jnp.maximum(f1, 0.0), w2_ref[...]) + b2_ref[...]
    hs = hs_ref[...]
    score_ref[...] = jnp.sum(hs * f, axis=1, keepdims=True)
    t_ref[...] = hs + f


def _edge_pass(hsrc, vecT, pfA, p, lidx):
    E = hsrc.shape[0]
    rbfw = p["rbf_W"].reshape(RBF_K, 1)
    w1 = p["frame_W1"]
    w1r = w1[:, :RBF_K]
    w1p = jnp.pad(w1[:, RBF_K:RBF_K + POS_EMB],
                  ((0, 0), (16 * lidx, 48 - 16 * lidx - 16)))
    w1d = jnp.pad(w1[:, RBF_K + POS_EMB:], ((0, 0), (0, 1)))
    b1 = p["frame_b1"].reshape(1, H)
    w2 = p["frame_W2"]
    b2 = p["frame_b2"].reshape(1, H)
    grid = E // EB
    t, score = pl.pallas_call(
        _edge_body,
        grid=(grid,),
        in_specs=[
            pl.BlockSpec((EB, H), lambda i: (i, 0)),
            pl.BlockSpec((4, EB), lambda i: (0, i)),
            pl.BlockSpec((EB, 48), lambda i: (i, 0)),
            pl.BlockSpec((RBF_K, 1), lambda i: (0, 0)),
            pl.BlockSpec((H, RBF_K), lambda i: (0, 0)),
            pl.BlockSpec((H, 48), lambda i: (0, 0)),
            pl.BlockSpec((H, 4), lambda i: (0, 0)),
            pl.BlockSpec((1, H), lambda i: (0, 0)),
            pl.BlockSpec((H, H), lambda i: (0, 0)),
            pl.BlockSpec((1, H), lambda i: (0, 0)),
        ],
        out_specs=[
            pl.BlockSpec((EB, H), lambda i: (i, 0)),
            pl.BlockSpec((EB, 1), lambda i: (i, 0)),
        ],
        out_shape=[
            jax.ShapeDtypeStruct((E, H), _f32),
            jax.ShapeDtypeStruct((E, 1), _f32),
        ],
    )(hsrc, vecT, pfA, rbfw, w1r, w1p, w1d, b1, w2, b2)
    return t, score.reshape(E)


def _nk0_body(x_ref, w_ref, b_ref, xw_ref, xb_ref, h_ref, hp_ref):
    hv = _mm_t(x_ref[...], w_ref[...]) + b_ref[...]
    h_ref[...] = hv
    hp_ref[...] = _mm_t(hv, xw_ref[...]) + xb_ref[...]


def _nk0(x, params):
    N = x.shape[0]
    p0 = params["layers"][0]
    return pl.pallas_call(
        _nk0_body,
        grid=(N // BN,),
        in_specs=[
            pl.BlockSpec((BN, H), lambda i: (i, 0)),
            pl.BlockSpec((H, H), lambda i: (0, 0)),
            pl.BlockSpec((1, H), lambda i: (0, 0)),
            pl.BlockSpec((H, H), lambda i: (0, 0)),
            pl.BlockSpec((1, H), lambda i: (0, 0)),
        ],
        out_specs=[
            pl.BlockSpec((BN, H), lambda i: (i, 0)),
            pl.BlockSpec((BN, H), lambda i: (i, 0)),
        ],
        out_shape=[
            jax.ShapeDtypeStruct((N, H), _f32),
            jax.ShapeDtypeStruct((N, H), _f32),
        ],
    )(x, params["in_W"], params["in_b"].reshape(1, H),
      p0["xp_W"], p0["xp_b"].reshape(1, H))


def _nku_body_proj(h_ref, hp_ref, a0_ref, a1_ref, d2_ref, g1a_ref, g1b_ref,
                   gb1_ref, g2_ref, gb2_ref, uw_ref, ub_ref, lng_ref, lnb_ref,
                   xw_ref, xb_ref, hn_ref, hpn_ref):
    hn = _nku_common(h_ref, hp_ref, a0_ref, a1_ref, d2_ref, g1a_ref, g1b_ref,
                     gb1_ref, g2_ref, gb2_ref, uw_ref, ub_ref, lng_ref,
                     lnb_ref)
    hn_ref[...] = hn
    hpn_ref[...] = _mm_t(hn, xw_ref[...]) + xb_ref[...]


def _nku_body_last(h_ref, hp_ref, a0_ref, a1_ref, d2_ref, g1a_ref, g1b_ref,
                   gb1_ref, g2_ref, gb2_ref, uw_ref, ub_ref, lng_ref, lnb_ref,
                   hn_ref):
    hn_ref[...] = _nku_common(h_ref, hp_ref, a0_ref, a1_ref, d2_ref, g1a_ref,
                              g1b_ref, gb1_ref, g2_ref, gb2_ref, uw_ref,
                              ub_ref, lng_ref, lnb_ref)


def _nku_common(h_ref, hp_ref, a0_ref, a1_ref, d2_ref, g1a_ref, g1b_ref,
                gb1_ref, g2_ref, gb2_ref, uw_ref, ub_ref, lng_ref, lnb_ref):
    ones_h1 = jnp.ones((H, 1), _f32)
    a = a0_ref[0] + a1_ref[0]
    dcol = lax.dot_general(d2_ref[...], jnp.ones((1, NWORK), _f32),
                           (((0,), (1,)), ((), ())),
                           preferred_element_type=_f32)
    inv_b = _mm_t(1.0 / (dcol + 1e-16), ones_h1)
    aggr = a * inv_b
    hpv = hp_ref[...]
    gi1 = (_mm_t(aggr, g1a_ref[...]) + _mm_t(hpv, g1b_ref[...])
           + gb1_ref[...])
    gate = jax.nn.sigmoid(_mm_t(jnp.maximum(gi1, 0.0), g2_ref[...])
                          + gb2_ref[...])
    gated = gate * aggr + (1.0 - gate) * hpv
    out = jnp.maximum(_mm_t(gated, uw_ref[...]) + ub_ref[...], 0.0)
    v = out + h_ref[...]
    ones_hh = jnp.full((H, H), 1.0 / H, _f32)
    mu_b = _mm_t(v, ones_hh)
    cen = v - mu_b
    var_b = _mm_t(cen * cen, ones_hh)
    return cen / jnp.sqrt(var_b + 1e-5) * lng_ref[...] + lnb_ref[...]


def _nku(h, hp, aggr2, den2, p, pnext):
    N = h.shape[0]
    last = pnext is None
    body = _nku_body_last if last else _nku_body_proj
    in_specs = [
        pl.BlockSpec((BN, H), lambda i: (i, 0)),
        pl.BlockSpec((BN, H), lambda i: (i, 0)),
        pl.BlockSpec((1, BN, H), lambda i: (0, i, 0)),
        pl.BlockSpec((1, BN, H), lambda i: (1, i, 0)),
        pl.BlockSpec((NWORK, BN), lambda i: (0, i)),
        pl.BlockSpec((H, H), lambda i: (0, 0)),
        pl.BlockSpec((H, H), lambda i: (0, 0)),
        pl.BlockSpec((1, H), lambda i: (0, 0)),
        pl.BlockSpec((H, H), lambda i: (0, 0)),
        pl.BlockSpec((1, H), lambda i: (0, 0)),
        pl.BlockSpec((H, H), lambda i: (0, 0)),
        pl.BlockSpec((1, H), lambda i: (0, 0)),
        pl.BlockSpec((1, H), lambda i: (0, 0)),
        pl.BlockSpec((1, H), lambda i: (0, 0)),
    ]
    out_specs = [pl.BlockSpec((BN, H), lambda i: (i, 0))]
    out_shape = [jax.ShapeDtypeStruct((N, H), _f32)]
    args = [h, hp, aggr2, aggr2, den2,
            p["gate_W1"][:, :H], p["gate_W1"][:, H:],
            p["gate_b1"].reshape(1, H),
            jnp.broadcast_to(p["gate_W2"].reshape(1, H), (H, H)),
            jnp.broadcast_to(p["gate_b2"].reshape(1, 1), (1, H)),
            p["upd_W"], p["upd_b"].reshape(1, H),
            p["ln_g"].reshape(1, H), p["ln_b"].reshape(1, H)]
    if not last:
        in_specs += [pl.BlockSpec((H, H), lambda i: (0, 0)),
                     pl.BlockSpec((1, H), lambda i: (0, 0))]
        out_specs.append(pl.BlockSpec((BN, H), lambda i: (i, 0)))
        out_shape.append(jax.ShapeDtypeStruct((N, H), _f32))
        args += [pnext["xp_W"], pnext["xp_b"].reshape(1, H)]
    res = pl.pallas_call(
        body,
        grid=(N // BN,),
        in_specs=in_specs,
        out_specs=out_specs,
        out_shape=out_shape,
    )(*args)
    return (res[0], res[1]) if not last else (res[0], None)


def _readout_body(h_ref, st_ref, wv_ref, bv_ref, wo_ref, bo_ref, out_ref,
                  feats):
    for i in range(16):
        feats[pl.ds(i, 1), :] = h_ref[pl.ds(st_ref[i], 1), :]
    vv = _mm_t(feats[...], wv_ref[...]) + bv_ref[...]
    out_ref[...] = _mm_t(vv, wo_ref[...]) + bo_ref[...]


def _readout(h, starts, params):
    N = h.shape[0]
    return pl.pallas_call(
        _readout_body,
        in_specs=[
            pl.BlockSpec((N, H), lambda: (0, 0)),
            pl.BlockSpec(memory_space=pltpu.SMEM),
            pl.BlockSpec((H, H), lambda: (0, 0)),
            pl.BlockSpec((1, H), lambda: (0, 0)),
            pl.BlockSpec((H, H), lambda: (0, 0)),
            pl.BlockSpec((1, H), lambda: (0, 0)),
        ],
        out_specs=pl.BlockSpec((16, H), lambda: (0, 0)),
        out_shape=jax.ShapeDtypeStruct((16, H), _f32),
        scratch_shapes=[pltpu.VMEM((16, H), _f32)],
    )(h, starts, params["mha_Wv"], params["mha_bv"].reshape(1, H),
      params["mha_Wo"], params["mha_bo"].reshape(1, H))


def kernel(x, pos, params, edge_index, num_aa):
    N = x.shape[0]
    E = edge_index.shape[1]
    src = edge_index[0]
    dst = edge_index[1]
    pos4 = jnp.pad(pos, ((0, 0), (0, 1)))
    x = jnp.pad(x, ((0, NP - N), (0, 0)))

    prologue = _make_prologue(E, N)
    gather_rows = _make_gather_rows(E, N)
    segmax = _make_segmax(E)
    segdenagg = _make_segdenagg(E)

    posf = pos4.reshape(N * 4)
    pef = jnp.concatenate([p["pos_emb"] for p in params["layers"]],
                          axis=1).reshape(512 * 48)
    vecf, pff = prologue(posf, src, dst, pef)
    vecT = vecf.reshape(4, E)
    pfA = pff.reshape(E, 48)
    h, hp = _nk0(x, params)
    nl = len(params["layers"])
    for l, p in enumerate(params["layers"]):
        hsrc = gather_rows(hp, src)
        t, score = _edge_pass(hsrc, vecT, pfA, p, l)
        m2 = segmax(score, dst)
        den2, aggr2 = segdenagg(m2, score, dst, t)
        pnext = params["layers"][l + 1] if l + 1 < nl else None
        h, hp = _nku(h, hp, aggr2.reshape(2, NP, H),
                     den2.reshape(NWORK, NP), p, pnext)

    starts = jnp.cumsum(num_aa) - num_aa
    return _readout(h, starts.astype(_i32), params)


# speedup vs baseline: 1.3797x; 1.0275x over previous
import functools

import jax
import jax.numpy as jnp
from jax import lax
from jax.experimental import pallas as pl
from jax.experimental.pallas import tpu as pltpu
from jax.experimental.pallas import tpu_sc as plsc

H = 128
RBF_K = 16
POS_EMB = 16
EB = 3200
BN = 2048
NP = 10240
NSL = NP // 16
NWORK = 32
RC = 80
SCH = 2000

_f32 = jnp.float32
_i32 = jnp.int32


def _mesh():
    return plsc.VectorSubcoreMesh(core_axis_name="c", subcore_axis_name="s")


def _wid():
    return lax.axis_index("s") * 2 + lax.axis_index("c")


def _mm_t(a, b):
    return lax.dot_general(a, b, (((1,), (1,)), ((), ())),
                           preferred_element_type=_f32)


def _make_prologue(E, N):
    TPE = E // NWORK
    NCH = TPE // RC

    @functools.partial(
        pl.kernel,
        out_type=[jax.ShapeDtypeStruct((4 * E,), _f32),
                  jax.ShapeDtypeStruct((48 * E,), _f32)],
        mesh=_mesh(),
        compiler_params=pltpu.CompilerParams(needs_layout_passes=False),
        scratch_types=[pltpu.VMEM((N * 4,), _f32),
                       pltpu.VMEM((512 * 48,), _f32),
                       pltpu.VMEM((RC,), _i32),
                       pltpu.VMEM((RC,), _i32),
                       pltpu.VMEM((RC,), _i32),
                       pltpu.VMEM((4 * RC,), _f32),
                       pltpu.VMEM((48 * RC,), _f32)],
    )
    def k(posf, srcr, dstr, pef, vec_o, pf_o,
          posv, pev, sbuf, dbuf, relb, vecb, pfb):
        base = _wid() * TPE
        iota = lax.iota(_i32, 16)
        pltpu.sync_copy(posf, posv)
        pltpu.sync_copy(pef, pev)

        def chunk(kk, _):
            off = base + kk * RC
            pltpu.sync_copy(srcr.at[pl.ds(off, RC)], sbuf)
            pltpu.sync_copy(dstr.at[pl.ds(off, RC)], dbuf)

            def vr(i, _):
                sl = pl.ds(i * 16, 16)
                s16 = sbuf[sl]
                d16 = dbuf[sl]
                s4 = s16 * 4
                d4 = d16 * 4
                for c in range(3):
                    a = plsc.load_gather(posv, [s4 + c])
                    b = plsc.load_gather(posv, [d4 + c])
                    vecb[pl.ds(c * RC + i * 16, 16)] = a - b
                vecb[pl.ds(3 * RC + i * 16, 16)] = jnp.zeros((16,), _f32)
                relb[sl] = jnp.clip(d16 - s16, 0, 511) * 48
                return 0

            lax.fori_loop(0, RC // 16, vr, 0)

            def edge(e, _):
                r16 = plsc.load_gather(relb, [jnp.full((16,), e, _i32)])
                for l in range(3):
                    pfb[pl.ds(e * 48 + l * 16, 16)] = plsc.load_gather(
                        pev, [r16 + l * 16 + iota])
                return 0

            lax.fori_loop(0, RC, edge, 0)
            for c in range(4):
                pltpu.sync_copy(vecb.at[pl.ds(c * RC, RC)],
                                vec_o.at[pl.ds(c * E + off, RC)])
            pltpu.sync_copy(pfb, pf_o.at[pl.ds(off * 48, RC * 48)])
            return 0

        lax.fori_loop(0, NCH, chunk, 0)

    return k


def _make_gather_rows(E, N):
    TPE = E // NWORK
    NCH = TPE // RC

    NBUF = 5

    @functools.partial(
        pl.kernel,
        out_type=jax.ShapeDtypeStruct((E, H), _f32),
        mesh=_mesh(),
        compiler_params=pltpu.CompilerParams(needs_layout_passes=False),
        scratch_types=[pltpu.VMEM((TPE,), _i32)]
        + [pltpu.VMEM((RC, H), _f32) for _ in range(NBUF)]
        + [pltpu.SemaphoreType.DMA for _ in range(NBUF)],
    )
    def k(hp, srcr, out, ibuf, *bufsem):
        rbs = bufsem[:NBUF]
        sems = bufsem[NBUF:]
        base = _wid() * TPE
        pltpu.sync_copy(srcr.at[pl.ds(base, TPE)], ibuf)

        def group(kk, _):
            g0 = kk * (NBUF * RC)
            cps = [
                pltpu.async_copy(
                    hp.at[ibuf.at[pl.ds(g0 + j * RC, RC)]], rbs[j], sems[j])
                for j in range(NBUF)
            ]
            for j in range(NBUF):
                cps[j].wait()
                pltpu.sync_copy(rbs[j],
                                out.at[pl.ds(base + g0 + j * RC, RC)])
            return 0

        lax.fori_loop(0, NCH // NBUF, group, 0)

    return k


def _make_segmax(E):
    TPE = E // NWORK
    NCH = TPE // SCH

    @functools.partial(
        pl.kernel,
        out_type=jax.ShapeDtypeStruct((2 * NP,), _f32),
        mesh=_mesh(),
        compiler_params=pltpu.CompilerParams(needs_layout_passes=False),
        scratch_types=[pltpu.VMEM((NP,), _f32),
                       pltpu.VMEM((SCH,), _i32),
                       pltpu.VMEM((SCH,), _f32),
                       pltpu.VMEM((NSL,), _f32),
                       pltpu.VMEM((NSL,), _f32),
                       pltpu.VMEM_SHARED((16 * NP,), _f32)],
    )
    def k(score, dstr, m2, mloc, dbuf, sbuf, acc, tmp, shm):
        cid = lax.axis_index("c")
        sid = lax.axis_index("s")
        base = _wid() * TPE
        iota = lax.iota(_i32, 16)

        def z(i, _):
            mloc[pl.ds(i * 16, 16)] = jnp.full((16,), -1e30, _f32)
            return 0

        lax.fori_loop(0, NP // 16, z, 0)

        def chunk(kk, _):
            off = base + kk * SCH
            pltpu.sync_copy(dstr.at[pl.ds(off, SCH)], dbuf)
            pltpu.sync_copy(score.at[pl.ds(off, SCH)], sbuf)

            def vr(i, _):
                d0 = dbuf[pl.ds(i * 16, 16)]
                s0 = sbuf[pl.ds(i * 16, 16)]
                d, s = plsc.sort_key_val(d0, s0)
                run = s
                for ksh in (1, 2, 4, 8):
                    idx = jnp.maximum(iota - ksh, 0)
                    dsh = d.at[idx].get(mode="promise_in_bounds")
                    rsh = run.at[idx].get(mode="promise_in_bounds")
                    ok = (iota >= ksh) & (dsh == d)
                    run = jnp.where(ok, jnp.maximum(run, rsh), run)
                nxt = jnp.minimum(iota + 1, 15)
                dn = d.at[nxt].get(mode="promise_in_bounds")
                last = (dn != d) | (iota == 15)
                cur = plsc.load_gather(mloc, [d])
                plsc.store_scatter(mloc, [d], jnp.maximum(cur, run),
                                   mask=last)
                return 0

            lax.fori_loop(0, SCH // 16, vr, 0)
            return 0

        lax.fori_loop(0, NCH, chunk, 0)

        pltpu.sync_copy(mloc, shm.at[pl.ds(sid * NP, NP)])
        plsc.subcore_barrier()
        s0 = sid * NSL
        pltpu.sync_copy(shm.at[pl.ds(s0, NSL)], acc)
        for r in range(1, 16):
            pltpu.sync_copy(shm.at[pl.ds(r * NP + s0, NSL)], tmp)

            def mx(i, _):
                acc[pl.ds(i * 16, 16)] = jnp.maximum(
                    acc[pl.ds(i * 16, 16)], tmp[pl.ds(i * 16, 16)])
                return 0

            lax.fori_loop(0, NSL // 16, mx, 0)
        pltpu.sync_copy(acc, m2.at[pl.ds(cid * NP + s0, NSL)])

    return k


def _make_segdenagg(E):
    TPE = E // NWORK
    NCH = TPE // RC
    ZR = 64

    @functools.partial(
        pl.kernel,
        out_type=[jax.ShapeDtypeStruct((NWORK * NP,), _f32),
                  jax.ShapeDtypeStruct((2 * NP, H), _f32)],
        mesh=_mesh(),
        compiler_params=pltpu.CompilerParams(needs_layout_passes=False),
        scratch_types=[pltpu.VMEM((NP,), _f32),
                       pltpu.VMEM((NP,), _f32),
                       pltpu.VMEM((RC, H), _f32),
                       pltpu.VMEM((RC, H), _f32),
                       pltpu.VMEM((RC,), _i32),
                       pltpu.VMEM((RC,), _i32),
                       pltpu.VMEM((RC,), _f32),
                       pltpu.VMEM((RC,), _f32),
                       pltpu.VMEM_SHARED((NP, H), _f32),
                       pltpu.SemaphoreType.DMA,
                       pltpu.SemaphoreType.DMA,
                       pltpu.SemaphoreType.DMA,
                       pltpu.SemaphoreType.DMA],
    )
    def k(m2, score, dstr, t, den2, aggr2, ma, den, tb0, tb1, ib0, ib1,
          sb0, sb1, ash, sm0, sm1, sa0, sa1):
        cid = lax.axis_index("c")
        sid = lax.axis_index("s")
        base = _wid() * TPE
        s0 = sid * NSL
        pltpu.sync_copy(m2.at[pl.ds(0, NP)], ma)
        pltpu.sync_copy(m2.at[pl.ds(NP, NP)], den)

        def mx(i, _):
            ma[pl.ds(i * 16, 16)] = jnp.maximum(ma[pl.ds(i * 16, 16)],
                                                den[pl.ds(i * 16, 16)])
            return 0

        lax.fori_loop(0, NP // 16, mx, 0)

        def z(i, _):
            den[pl.ds(i * 16, 16)] = jnp.zeros((16,), _f32)
            return 0

        lax.fori_loop(0, NP // 16, z, 0)

        def zz(i, _):
            for j in range(H // 16):
                tb0[i, pl.ds(j * 16, 16)] = jnp.zeros((16,), _f32)
            return 0

        lax.fori_loop(0, RC, zz, 0)
        for j in range(NSL // RC):
            pltpu.sync_copy(tb0, ash.at[pl.ds(s0 + j * RC, RC)])
        plsc.subcore_barrier()

        def half(ib, sb, tb, cp, sa):
            def ex(i, _):
                d = ib[pl.ds(i * 16, 16)]
                mv = plsc.load_gather(ma, [d])
                e16 = jnp.exp(sb[pl.ds(i * 16, 16)] - mv)
                sb[pl.ds(i * 16, 16)] = e16
                plsc.addupdate_scatter(den, [d], e16)
                return 0

            lax.fori_loop(0, RC // 16, ex, 0)
            cp.wait()

            def row(e, _):
                ws = plsc.load_gather(sb, [jnp.full((16,), e, _i32)])
                for j in range(H // 16):
                    tb[e, pl.ds(j * 16, 16)] = tb[e, pl.ds(j * 16, 16)] * ws
                return 0

            lax.fori_loop(0, RC, row, 0)
            return pltpu.async_copy(tb, ash.at[ib], sa, add=True)

        def grp(kk, _):
            o0 = base + kk * 2 * RC
            c0 = pltpu.async_copy(t.at[pl.ds(o0, RC)], tb0, sm0)
            c1 = pltpu.async_copy(t.at[pl.ds(o0 + RC, RC)], tb1, sm1)
            pltpu.sync_copy(dstr.at[pl.ds(o0, RC)], ib0)
            pltpu.sync_copy(score.at[pl.ds(o0, RC)], sb0)
            pltpu.sync_copy(dstr.at[pl.ds(o0 + RC, RC)], ib1)
            pltpu.sync_copy(score.at[pl.ds(o0 + RC, RC)], sb1)
            a0 = half(ib0, sb0, tb0, c0, sa0)
            a1 = half(ib1, sb1, tb1, c1, sa1)
            a0.wait()
            a1.wait()
            return 0

        lax.fori_loop(0, NCH // 2, grp, 0)

        if NCH % 2:
            oT = base + (NCH - 1) * RC
            cT = pltpu.async_copy(t.at[pl.ds(oT, RC)], tb0, sm0)
            pltpu.sync_copy(dstr.at[pl.ds(oT, RC)], ib0)
            pltpu.sync_copy(score.at[pl.ds(oT, RC)], sb0)
            half(ib0, sb0, tb0, cT, sa0).wait()

        pltpu.sync_copy(den, den2.at[pl.ds(_wid() * NP, NP)])
        plsc.subcore_barrier()
        pltpu.sync_copy(ash.at[pl.ds(s0, NSL)],
                        aggr2.at[pl.ds(cid * NP + s0, NSL)])

    return k


def _edge_body(hs_ref, vec_ref, pf_ref, rbfw_ref, w1r_ref, w1p_ref,
               w1d_ref, b1_ref, w2_ref, b2_ref, t_ref, score_ref):
    v4 = vec_ref[...]
    d2r = jnp.sum(v4 * v4, axis=0, keepdims=True)
    dcol = lax.dot_general(d2r, jnp.ones((1, 1), _f32),
                           (((0,), (1,)), ((), ())),
                           preferred_element_type=_f32)
    dist_col = jnp.sqrt(dcol)
    rbf = jnp.exp(-jnp.square(_mm_t(dist_col, rbfw_ref[...])))
    invdr = 1.0 / jnp.clip(jnp.sqrt(d2r), 1e-12, None)
    dir4 = v4 * invdr
    f_dir = lax.dot_general(dir4, w1d_ref[...],
                            (((0,), (1,)), ((), ())),
                            preferred_element_type=_f32)
    f_pos = _mm_t(pf_ref[...], w1p_ref[...])
    f1 = _mm_t(rbf, w1r_ref[...]) + f_pos + f_dir + b1_ref[...]
    f = _mm_t(jnp.maximum(f1, 0.0), w2_ref[...]) + b2_ref[...]
    hs = hs_ref[...]
    score_ref[...] = jnp.sum(hs * f, axis=1, keepdims=True)
    t_ref[...] = hs + f


def _edge_pass(hsrc, vecT, pfA, p, lidx):
    E = hsrc.shape[0]
    rbfw = p["rbf_W"].reshape(RBF_K, 1)
    w1 = p["frame_W1"]
    w1r = w1[:, :RBF_K]
    w1p = jnp.pad(w1[:, RBF_K:RBF_K + POS_EMB],
                  ((0, 0), (16 * lidx, 48 - 16 * lidx - 16)))
    w1d = jnp.pad(w1[:, RBF_K + POS_EMB:], ((0, 0), (0, 1)))
    b1 = p["frame_b1"].reshape(1, H)
    w2 = p["frame_W2"]
    b2 = p["frame_b2"].reshape(1, H)
    grid = E // EB
    t, score = pl.pallas_call(
        _edge_body,
        grid=(grid,),
        in_specs=[
            pl.BlockSpec((EB, H), lambda i: (i, 0)),
            pl.BlockSpec((4, EB), lambda i: (0, i)),
            pl.BlockSpec((EB, 48), lambda i: (i, 0)),
            pl.BlockSpec((RBF_K, 1), lambda i: (0, 0)),
            pl.BlockSpec((H, RBF_K), lambda i: (0, 0)),
            pl.BlockSpec((H, 48), lambda i: (0, 0)),
            pl.BlockSpec((H, 4), lambda i: (0, 0)),
            pl.BlockSpec((1, H), lambda i: (0, 0)),
            pl.BlockSpec((H, H), lambda i: (0, 0)),
            pl.BlockSpec((1, H), lambda i: (0, 0)),
        ],
        out_specs=[
            pl.BlockSpec((EB, H), lambda i: (i, 0)),
            pl.BlockSpec((EB, 1), lambda i: (i, 0)),
        ],
        out_shape=[
            jax.ShapeDtypeStruct((E, H), _f32),
            jax.ShapeDtypeStruct((E, 1), _f32),
        ],
    )(hsrc, vecT, pfA, rbfw, w1r, w1p, w1d, b1, w2, b2)
    return t, score.reshape(E)


def _nk0_body(x_ref, w_ref, b_ref, xw_ref, xb_ref, h_ref, hp_ref):
    hv = _mm_t(x_ref[...], w_ref[...]) + b_ref[...]
    h_ref[...] = hv
    hp_ref[...] = _mm_t(hv, xw_ref[...]) + xb_ref[...]


def _nk0(x, params):
    N = x.shape[0]
    p0 = params["layers"][0]
    return pl.pallas_call(
        _nk0_body,
        grid=(N // BN,),
        in_specs=[
            pl.BlockSpec((BN, H), lambda i: (i, 0)),
            pl.BlockSpec((H, H), lambda i: (0, 0)),
            pl.BlockSpec((1, H), lambda i: (0, 0)),
            pl.BlockSpec((H, H), lambda i: (0, 0)),
            pl.BlockSpec((1, H), lambda i: (0, 0)),
        ],
        out_specs=[
            pl.BlockSpec((BN, H), lambda i: (i, 0)),
            pl.BlockSpec((BN, H), lambda i: (i, 0)),
        ],
        out_shape=[
            jax.ShapeDtypeStruct((N, H), _f32),
            jax.ShapeDtypeStruct((N, H), _f32),
        ],
    )(x, params["in_W"], params["in_b"].reshape(1, H),
      p0["xp_W"], p0["xp_b"].reshape(1, H))


def _nku_body_proj(h_ref, hp_ref, a0_ref, a1_ref, d2_ref, g1a_ref, g1b_ref,
                   gb1_ref, g2_ref, gb2_ref, uw_ref, ub_ref, lng_ref, lnb_ref,
                   xw_ref, xb_ref, hn_ref, hpn_ref):
    hn = _nku_common(h_ref, hp_ref, a0_ref, a1_ref, d2_ref, g1a_ref, g1b_ref,
                     gb1_ref, g2_ref, gb2_ref, uw_ref, ub_ref, lng_ref,
                     lnb_ref)
    hn_ref[...] = hn
    hpn_ref[...] = _mm_t(hn, xw_ref[...]) + xb_ref[...]


def _nku_body_last(h_ref, hp_ref, a0_ref, a1_ref, d2_ref, g1a_ref, g1b_ref,
                   gb1_ref, g2_ref, gb2_ref, uw_ref, ub_ref, lng_ref, lnb_ref,
                   hn_ref):
    hn_ref[...] = _nku_common(h_ref, hp_ref, a0_ref, a1_ref, d2_ref, g1a_ref,
                              g1b_ref, gb1_ref, g2_ref, gb2_ref, uw_ref,
                              ub_ref, lng_ref, lnb_ref)


def _nku_common(h_ref, hp_ref, a0_ref, a1_ref, d2_ref, g1a_ref, g1b_ref,
                gb1_ref, g2_ref, gb2_ref, uw_ref, ub_ref, lng_ref, lnb_ref):
    ones_h1 = jnp.ones((H, 1), _f32)
    a = a0_ref[0] + a1_ref[0]
    dcol = lax.dot_general(d2_ref[...], jnp.ones((1, NWORK), _f32),
                           (((0,), (1,)), ((), ())),
                           preferred_element_type=_f32)
    inv_b = _mm_t(1.0 / (dcol + 1e-16), ones_h1)
    aggr = a * inv_b
    hpv = hp_ref[...]
    gi1 = (_mm_t(aggr, g1a_ref[...]) + _mm_t(hpv, g1b_ref[...])
           + gb1_ref[...])
    gate = jax.nn.sigmoid(_mm_t(jnp.maximum(gi1, 0.0), g2_ref[...])
                          + gb2_ref[...])
    gated = gate * aggr + (1.0 - gate) * hpv
    out = jnp.maximum(_mm_t(gated, uw_ref[...]) + ub_ref[...], 0.0)
    v = out + h_ref[...]
    ones_hh = jnp.full((H, H), 1.0 / H, _f32)
    mu_b = _mm_t(v, ones_hh)
    cen = v - mu_b
    var_b = _mm_t(cen * cen, ones_hh)
    return cen / jnp.sqrt(var_b + 1e-5) * lng_ref[...] + lnb_ref[...]


def _nku(h, hp, aggr2, den2, p, pnext):
    N = h.shape[0]
    last = pnext is None
    body = _nku_body_last if last else _nku_body_proj
    in_specs = [
        pl.BlockSpec((BN, H), lambda i: (i, 0)),
        pl.BlockSpec((BN, H), lambda i: (i, 0)),
        pl.BlockSpec((1, BN, H), lambda i: (0, i, 0)),
        pl.BlockSpec((1, BN, H), lambda i: (1, i, 0)),
        pl.BlockSpec((NWORK, BN), lambda i: (0, i)),
        pl.BlockSpec((H, H), lambda i: (0, 0)),
        pl.BlockSpec((H, H), lambda i: (0, 0)),
        pl.BlockSpec((1, H), lambda i: (0, 0)),
        pl.BlockSpec((H, H), lambda i: (0, 0)),
        pl.BlockSpec((1, H), lambda i: (0, 0)),
        pl.BlockSpec((H, H), lambda i: (0, 0)),
        pl.BlockSpec((1, H), lambda i: (0, 0)),
        pl.BlockSpec((1, H), lambda i: (0, 0)),
        pl.BlockSpec((1, H), lambda i: (0, 0)),
    ]
    out_specs = [pl.BlockSpec((BN, H), lambda i: (i, 0))]
    out_shape = [jax.ShapeDtypeStruct((N, H), _f32)]
    args = [h, hp, aggr2, aggr2, den2,
            p["gate_W1"][:, :H], p["gate_W1"][:, H:],
            p["gate_b1"].reshape(1, H),
            jnp.broadcast_to(p["gate_W2"].reshape(1, H), (H, H)),
            jnp.broadcast_to(p["gate_b2"].reshape(1, 1), (1, H)),
            p["upd_W"], p["upd_b"].reshape(1, H),
            p["ln_g"].reshape(1, H), p["ln_b"].reshape(1, H)]
    if not last:
        in_specs += [pl.BlockSpec((H, H), lambda i: (0, 0)),
                     pl.BlockSpec((1, H), lambda i: (0, 0))]
        out_specs.append(pl.BlockSpec((BN, H), lambda i: (i, 0)))
        out_shape.append(jax.ShapeDtypeStruct((N, H), _f32))
        args += [pnext["xp_W"], pnext["xp_b"].reshape(1, H)]
    res = pl.pallas_call(
        body,
        grid=(N // BN,),
        in_specs=in_specs,
        out_specs=out_specs,
        out_shape=out_shape,
    )(*args)
    return (res[0], res[1]) if not last else (res[0], None)


def _readout_body(h_ref, st_ref, wv_ref, bv_ref, wo_ref, bo_ref, out_ref,
                  feats):
    for i in range(16):
        feats[pl.ds(i, 1), :] = h_ref[pl.ds(st_ref[i], 1), :]
    vv = _mm_t(feats[...], wv_ref[...]) + bv_ref[...]
    out_ref[...] = _mm_t(vv, wo_ref[...]) + bo_ref[...]


def _readout(h, starts, params):
    N = h.shape[0]
    return pl.pallas_call(
        _readout_body,
        in_specs=[
            pl.BlockSpec((N, H), lambda: (0, 0)),
            pl.BlockSpec(memory_space=pltpu.SMEM),
            pl.BlockSpec((H, H), lambda: (0, 0)),
            pl.BlockSpec((1, H), lambda: (0, 0)),
            pl.BlockSpec((H, H), lambda: (0, 0)),
            pl.BlockSpec((1, H), lambda: (0, 0)),
        ],
        out_specs=pl.BlockSpec((16, H), lambda: (0, 0)),
        out_shape=jax.ShapeDtypeStruct((16, H), _f32),
        scratch_shapes=[pltpu.VMEM((16, H), _f32)],
    )(h, starts, params["mha_Wv"], params["mha_bv"].reshape(1, H),
      params["mha_Wo"], params["mha_bo"].reshape(1, H))


def kernel(x, pos, params, edge_index, num_aa):
    N = x.shape[0]
    E = edge_index.shape[1]
    src = edge_index[0]
    dst = edge_index[1]
    pos4 = jnp.pad(pos, ((0, 0), (0, 1)))
    x = jnp.pad(x, ((0, NP - N), (0, 0)))

    prologue = _make_prologue(E, N)
    gather_rows = _make_gather_rows(E, N)
    segmax = _make_segmax(E)
    segdenagg = _make_segdenagg(E)

    posf = pos4.reshape(N * 4)
    pef = jnp.concatenate([p["pos_emb"] for p in params["layers"]],
                          axis=1).reshape(512 * 48)
    vecf, pff = prologue(posf, src, dst, pef)
    vecT = vecf.reshape(4, E)
    pfA = pff.reshape(E, 48)
    h, hp = _nk0(x, params)
    nl = len(params["layers"])
    for l, p in enumerate(params["layers"]):
        hsrc = gather_rows(hp, src)
        t, score = _edge_pass(hsrc, vecT, pfA, p, l)
        m2 = segmax(score, dst)
        den2, aggr2 = segdenagg(m2, score, dst, t)
        pnext = params["layers"][l + 1] if l + 1 < nl else None
        h, hp = _nku(h, hp, aggr2.reshape(2, NP, H),
                     den2.reshape(NWORK, NP), p, pnext)

    starts = jnp.cumsum(num_aa) - num_aa
    return _readout(h, starts.astype(_i32), params)


# speedup vs baseline: 1.4468x; 1.0487x over previous
import functools

import jax
import jax.numpy as jnp
from jax import lax
from jax.experimental import pallas as pl
from jax.experimental.pallas import tpu as pltpu
from jax.experimental.pallas import tpu_sc as plsc

H = 128
RBF_K = 16
POS_EMB = 16
EB = 3200
BN = 2048
NP = 10240
NSL = NP // 16
NWORK = 32
RC = 80
SCH = 2000

_f32 = jnp.float32
_i32 = jnp.int32


def _mesh():
    return plsc.VectorSubcoreMesh(core_axis_name="c", subcore_axis_name="s")


def _wid():
    return lax.axis_index("s") * 2 + lax.axis_index("c")


def _mm_t(a, b):
    return lax.dot_general(a, b, (((1,), (1,)), ((), ())),
                           preferred_element_type=_f32)


def _make_prologue(E, N):
    TPE = E // NWORK
    EC = 400
    NG = TPE // EC

    @functools.partial(
        pl.kernel,
        out_type=[jax.ShapeDtypeStruct((4 * E,), _f32),
                  jax.ShapeDtypeStruct((48 * E,), _f32)],
        mesh=_mesh(),
        compiler_params=pltpu.CompilerParams(needs_layout_passes=False),
        scratch_types=[pltpu.VMEM((N * 4,), _f32),
                       pltpu.VMEM((512 * 48,), _f32),
                       pltpu.VMEM((TPE,), _i32),
                       pltpu.VMEM((TPE,), _i32),
                       pltpu.VMEM((EC,), _i32),
                       pltpu.VMEM((4 * EC,), _f32),
                       pltpu.VMEM((48 * EC,), _f32)],
    )
    def k(posf, srcr, dstr, pef, vec_o, pf_o,
          posv, pev, sV, dV, relb, vecb, pfb):
        base = _wid() * TPE
        iota = lax.iota(_i32, 16)
        pltpu.sync_copy(posf, posv)
        pltpu.sync_copy(pef, pev)
        pltpu.sync_copy(srcr.at[pl.ds(base, TPE)], sV)
        pltpu.sync_copy(dstr.at[pl.ds(base, TPE)], dV)

        def grp(g, _):
            off = base + g * EC

            def vr(i, _):
                gl = pl.ds(g * EC + i * 16, 16)
                sl = pl.ds(i * 16, 16)
                s16 = sV[gl]
                d16 = dV[gl]
                s4 = s16 * 4
                d4 = d16 * 4
                for c in range(3):
                    a = plsc.load_gather(posv, [s4 + c])
                    b = plsc.load_gather(posv, [d4 + c])
                    vecb[pl.ds(c * EC + i * 16, 16)] = a - b
                vecb[pl.ds(3 * EC + i * 16, 16)] = jnp.zeros((16,), _f32)
                relb[sl] = jnp.clip(d16 - s16, 0, 511) * 48
                return 0

            lax.fori_loop(0, EC // 16, vr, 0)

            def edge(e, _):
                r16 = plsc.load_gather(relb, [jnp.full((16,), e, _i32)])
                for l in range(3):
                    pfb[pl.ds(e * 48 + l * 16, 16)] = plsc.load_gather(
                        pev, [r16 + l * 16 + iota])
                return 0

            lax.fori_loop(0, EC, edge, 0)
            for c in range(4):
                pltpu.sync_copy(vecb.at[pl.ds(c * EC, EC)],
                                vec_o.at[pl.ds(c * E + off, EC)])
            pltpu.sync_copy(pfb, pf_o.at[pl.ds(off * 48, EC * 48)])
            return 0

        lax.fori_loop(0, NG, grp, 0)

    return k


def _make_gather_rows(E, N):
    TPE = E // NWORK
    NCH = TPE // RC

    NBUF = 5

    @functools.partial(
        pl.kernel,
        out_type=jax.ShapeDtypeStruct((E, H), _f32),
        mesh=_mesh(),
        compiler_params=pltpu.CompilerParams(needs_layout_passes=False),
        scratch_types=[pltpu.VMEM((TPE,), _i32)]
        + [pltpu.VMEM((RC, H), _f32) for _ in range(NBUF)]
        + [pltpu.SemaphoreType.DMA for _ in range(NBUF)],
    )
    def k(hp, srcr, out, ibuf, *bufsem):
        rbs = bufsem[:NBUF]
        sems = bufsem[NBUF:]
        base = _wid() * TPE
        pltpu.sync_copy(srcr.at[pl.ds(base, TPE)], ibuf)

        def group(kk, _):
            g0 = kk * (NBUF * RC)
            cps = [
                pltpu.async_copy(
                    hp.at[ibuf.at[pl.ds(g0 + j * RC, RC)]], rbs[j], sems[j])
                for j in range(NBUF)
            ]
            for j in range(NBUF):
                cps[j].wait()
                pltpu.sync_copy(rbs[j],
                                out.at[pl.ds(base + g0 + j * RC, RC)])
            return 0

        lax.fori_loop(0, NCH // NBUF, group, 0)

    return k


def _make_segmax(E):
    TPE = E // NWORK
    NCH = TPE // SCH

    @functools.partial(
        pl.kernel,
        out_type=jax.ShapeDtypeStruct((2 * NP,), _f32),
        mesh=_mesh(),
        compiler_params=pltpu.CompilerParams(needs_layout_passes=False),
        scratch_types=[pltpu.VMEM((NP,), _f32),
                       pltpu.VMEM((SCH,), _i32),
                       pltpu.VMEM((SCH,), _f32),
                       pltpu.VMEM((NSL,), _f32),
                       pltpu.VMEM((NSL,), _f32),
                       pltpu.VMEM_SHARED((16 * NP,), _f32)],
    )
    def k(score, dstr, m2, mloc, dbuf, sbuf, acc, tmp, shm):
        cid = lax.axis_index("c")
        sid = lax.axis_index("s")
        base = _wid() * TPE
        iota = lax.iota(_i32, 16)

        def z(i, _):
            mloc[pl.ds(i * 16, 16)] = jnp.full((16,), -1e30, _f32)
            return 0

        lax.fori_loop(0, NP // 16, z, 0)

        def chunk(kk, _):
            off = base + kk * SCH
            pltpu.sync_copy(dstr.at[pl.ds(off, SCH)], dbuf)
            pltpu.sync_copy(score.at[pl.ds(off, SCH)], sbuf)

            def vr(i, _):
                d0 = dbuf[pl.ds(i * 16, 16)]
                s0 = sbuf[pl.ds(i * 16, 16)]
                d, s = plsc.sort_key_val(d0, s0)
                run = s
                for ksh in (1, 2, 4, 8):
                    idx = jnp.maximum(iota - ksh, 0)
                    dsh = d.at[idx].get(mode="promise_in_bounds")
                    rsh = run.at[idx].get(mode="promise_in_bounds")
                    ok = (iota >= ksh) & (dsh == d)
                    run = jnp.where(ok, jnp.maximum(run, rsh), run)
                nxt = jnp.minimum(iota + 1, 15)
                dn = d.at[nxt].get(mode="promise_in_bounds")
                last = (dn != d) | (iota == 15)
                cur = plsc.load_gather(mloc, [d])
                plsc.store_scatter(mloc, [d], jnp.maximum(cur, run),
                                   mask=last)
                return 0

            lax.fori_loop(0, SCH // 16, vr, 0)
            return 0

        lax.fori_loop(0, NCH, chunk, 0)

        pltpu.sync_copy(mloc, shm.at[pl.ds(sid * NP, NP)])
        plsc.subcore_barrier()
        s0 = sid * NSL
        pltpu.sync_copy(shm.at[pl.ds(s0, NSL)], acc)
        for r in range(1, 16):
            pltpu.sync_copy(shm.at[pl.ds(r * NP + s0, NSL)], tmp)

            def mx(i, _):
                acc[pl.ds(i * 16, 16)] = jnp.maximum(
                    acc[pl.ds(i * 16, 16)], tmp[pl.ds(i * 16, 16)])
                return 0

            lax.fori_loop(0, NSL // 16, mx, 0)
        pltpu.sync_copy(acc, m2.at[pl.ds(cid * NP + s0, NSL)])

    return k


def _make_segdenagg(E):
    TPE = E // NWORK
    NCH = TPE // RC
    ZR = 64

    @functools.partial(
        pl.kernel,
        out_type=[jax.ShapeDtypeStruct((NWORK * NP,), _f32),
                  jax.ShapeDtypeStruct((2 * NP, H), _f32)],
        mesh=_mesh(),
        compiler_params=pltpu.CompilerParams(needs_layout_passes=False),
        scratch_types=[pltpu.VMEM((NP,), _f32),
                       pltpu.VMEM((NP,), _f32),
                       pltpu.VMEM((RC, H), _f32),
                       pltpu.VMEM((RC, H), _f32),
                       pltpu.VMEM((RC,), _i32),
                       pltpu.VMEM((RC,), _i32),
                       pltpu.VMEM((RC,), _f32),
                       pltpu.VMEM((RC,), _f32),
                       pltpu.VMEM_SHARED((NP, H), _f32),
                       pltpu.SemaphoreType.DMA,
                       pltpu.SemaphoreType.DMA,
                       pltpu.SemaphoreType.DMA,
                       pltpu.SemaphoreType.DMA],
    )
    def k(m2, score, dstr, t, den2, aggr2, ma, den, tb0, tb1, ib0, ib1,
          sb0, sb1, ash, sm0, sm1, sa0, sa1):
        cid = lax.axis_index("c")
        sid = lax.axis_index("s")
        base = _wid() * TPE
        s0 = sid * NSL
        pltpu.sync_copy(m2.at[pl.ds(0, NP)], ma)
        pltpu.sync_copy(m2.at[pl.ds(NP, NP)], den)

        def mx(i, _):
            ma[pl.ds(i * 16, 16)] = jnp.maximum(ma[pl.ds(i * 16, 16)],
                                                den[pl.ds(i * 16, 16)])
            return 0

        lax.fori_loop(0, NP // 16, mx, 0)

        def z(i, _):
            den[pl.ds(i * 16, 16)] = jnp.zeros((16,), _f32)
            return 0

        lax.fori_loop(0, NP // 16, z, 0)

        def zz(i, _):
            for j in range(H // 16):
                tb0[i, pl.ds(j * 16, 16)] = jnp.zeros((16,), _f32)
            return 0

        lax.fori_loop(0, RC, zz, 0)
        for j in range(NSL // RC):
            pltpu.sync_copy(tb0, ash.at[pl.ds(s0 + j * RC, RC)])
        plsc.subcore_barrier()

        def half(ib, sb, tb, cp, sa):
            def ex(i, _):
                d = ib[pl.ds(i * 16, 16)]
                mv = plsc.load_gather(ma, [d])
                e16 = jnp.exp(sb[pl.ds(i * 16, 16)] - mv)
                sb[pl.ds(i * 16, 16)] = e16
                plsc.addupdate_scatter(den, [d], e16)
                return 0

            lax.fori_loop(0, RC // 16, ex, 0)
            cp.wait()

            def row(e, _):
                ws = plsc.load_gather(sb, [jnp.full((16,), e, _i32)])
                for j in range(H // 16):
                    tb[e, pl.ds(j * 16, 16)] = tb[e, pl.ds(j * 16, 16)] * ws
                return 0

            lax.fori_loop(0, RC, row, 0)
            return pltpu.async_copy(tb, ash.at[ib], sa, add=True)

        def grp(kk, _):
            o0 = base + kk * 2 * RC
            c0 = pltpu.async_copy(t.at[pl.ds(o0, RC)], tb0, sm0)
            c1 = pltpu.async_copy(t.at[pl.ds(o0 + RC, RC)], tb1, sm1)
            pltpu.sync_copy(dstr.at[pl.ds(o0, RC)], ib0)
            pltpu.sync_copy(score.at[pl.ds(o0, RC)], sb0)
            pltpu.sync_copy(dstr.at[pl.ds(o0 + RC, RC)], ib1)
            pltpu.sync_copy(score.at[pl.ds(o0 + RC, RC)], sb1)
            a0 = half(ib0, sb0, tb0, c0, sa0)
            a1 = half(ib1, sb1, tb1, c1, sa1)
            a0.wait()
            a1.wait()
            return 0

        lax.fori_loop(0, NCH // 2, grp, 0)

        if NCH % 2:
            oT = base + (NCH - 1) * RC
            cT = pltpu.async_copy(t.at[pl.ds(oT, RC)], tb0, sm0)
            pltpu.sync_copy(dstr.at[pl.ds(oT, RC)], ib0)
            pltpu.sync_copy(score.at[pl.ds(oT, RC)], sb0)
            half(ib0, sb0, tb0, cT, sa0).wait()

        pltpu.sync_copy(den, den2.at[pl.ds(_wid() * NP, NP)])
        plsc.subcore_barrier()
        pltpu.sync_copy(ash.at[pl.ds(s0, NSL)],
                        aggr2.at[pl.ds(cid * NP + s0, NSL)])

    return k


def _edge_body(hs_ref, vec_ref, pf_ref, rbfw_ref, w1r_ref, w1p_ref,
               w1d_ref, b1_ref, w2_ref, b2_ref, t_ref, score_ref):
    v4 = vec_ref[...]
    d2r = jnp.sum(v4 * v4, axis=0, keepdims=True)
    dcol = lax.dot_general(d2r, jnp.ones((1, 1), _f32),
                           (((0,), (1,)), ((), ())),
                           preferred_element_type=_f32)
    dist_col = jnp.sqrt(dcol)
    rbf = jnp.exp(-jnp.square(_mm_t(dist_col, rbfw_ref[...])))
    invdr = 1.0 / jnp.clip(jnp.sqrt(d2r), 1e-12, None)
    dir4 = v4 * invdr
    f_dir = lax.dot_general(dir4, w1d_ref[...],
                            (((0,), (1,)), ((), ())),
                            preferred_element_type=_f32)
    f_pos = _mm_t(pf_ref[...], w1p_ref[...])
    f1 = _mm_t(rbf, w1r_ref[...]) + f_pos + f_dir + b1_ref[...]
    f = _mm_t(jnp.maximum(f1, 0.0), w2_ref[...]) + b2_ref[...]
    hs = hs_ref[...]
    score_ref[...] = jnp.sum(hs * f, axis=1, keepdims=True)
    t_ref[...] = hs + f


def _edge_pass(hsrc, vecT, pfA, p, lidx):
    E = hsrc.shape[0]
    rbfw = p["rbf_W"].reshape(RBF_K, 1)
    w1 = p["frame_W1"]
    w1r = w1[:, :RBF_K]
    w1p = jnp.pad(w1[:, RBF_K:RBF_K + POS_EMB],
                  ((0, 0), (16 * lidx, 48 - 16 * lidx - 16)))
    w1d = jnp.pad(w1[:, RBF_K + POS_EMB:], ((0, 0), (0, 1)))
    b1 = p["frame_b1"].reshape(1, H)
    w2 = p["frame_W2"]
    b2 = p["frame_b2"].reshape(1, H)
    grid = E // EB
    t, score = pl.pallas_call(
        _edge_body,
        grid=(grid,),
        in_specs=[
            pl.BlockSpec((EB, H), lambda i: (i, 0)),
            pl.BlockSpec((4, EB), lambda i: (0, i)),
            pl.BlockSpec((EB, 48), lambda i: (i, 0)),
            pl.BlockSpec((RBF_K, 1), lambda i: (0, 0)),
            pl.BlockSpec((H, RBF_K), lambda i: (0, 0)),
            pl.BlockSpec((H, 48), lambda i: (0, 0)),
            pl.BlockSpec((H, 4), lambda i: (0, 0)),
            pl.BlockSpec((1, H), lambda i: (0, 0)),
            pl.BlockSpec((H, H), lambda i: (0, 0)),
            pl.BlockSpec((1, H), lambda i: (0, 0)),
        ],
        out_specs=[
            pl.BlockSpec((EB, H), lambda i: (i, 0)),
            pl.BlockSpec((EB, 1), lambda i: (i, 0)),
        ],
        out_shape=[
            jax.ShapeDtypeStruct((E, H), _f32),
            jax.ShapeDtypeStruct((E, 1), _f32),
        ],
    )(hsrc, vecT, pfA, rbfw, w1r, w1p, w1d, b1, w2, b2)
    return t, score.reshape(E)


def _nk0_body(x_ref, w_ref, b_ref, xw_ref, xb_ref, h_ref, hp_ref):
    hv = _mm_t(x_ref[...], w_ref[...]) + b_ref[...]
    h_ref[...] = hv
    hp_ref[...] = _mm_t(hv, xw_ref[...]) + xb_ref[...]


def _nk0(x, params):
    N = x.shape[0]
    p0 = params["layers"][0]
    return pl.pallas_call(
        _nk0_body,
        grid=(N // BN,),
        in_specs=[
            pl.BlockSpec((BN, H), lambda i: (i, 0)),
            pl.BlockSpec((H, H), lambda i: (0, 0)),
            pl.BlockSpec((1, H), lambda i: (0, 0)),
            pl.BlockSpec((H, H), lambda i: (0, 0)),
            pl.BlockSpec((1, H), lambda i: (0, 0)),
        ],
        out_specs=[
            pl.BlockSpec((BN, H), lambda i: (i, 0)),
            pl.BlockSpec((BN, H), lambda i: (i, 0)),
        ],
        out_shape=[
            jax.ShapeDtypeStruct((N, H), _f32),
            jax.ShapeDtypeStruct((N, H), _f32),
        ],
    )(x, params["in_W"], params["in_b"].reshape(1, H),
      p0["xp_W"], p0["xp_b"].reshape(1, H))


def _nku_body_proj(h_ref, hp_ref, a0_ref, a1_ref, d2_ref, g1a_ref, g1b_ref,
                   gb1_ref, g2_ref, gb2_ref, uw_ref, ub_ref, lng_ref, lnb_ref,
                   xw_ref, xb_ref, hn_ref, hpn_ref):
    hn = _nku_common(h_ref, hp_ref, a0_ref, a1_ref, d2_ref, g1a_ref, g1b_ref,
                     gb1_ref, g2_ref, gb2_ref, uw_ref, ub_ref, lng_ref,
                     lnb_ref)
    hn_ref[...] = hn
    hpn_ref[...] = _mm_t(hn, xw_ref[...]) + xb_ref[...]


def _nku_body_last(h_ref, hp_ref, a0_ref, a1_ref, d2_ref, g1a_ref, g1b_ref,
                   gb1_ref, g2_ref, gb2_ref, uw_ref, ub_ref, lng_ref, lnb_ref,
                   hn_ref):
    hn_ref[...] = _nku_common(h_ref, hp_ref, a0_ref, a1_ref, d2_ref, g1a_ref,
                              g1b_ref, gb1_ref, g2_ref, gb2_ref, uw_ref,
                              ub_ref, lng_ref, lnb_ref)


def _nku_common(h_ref, hp_ref, a0_ref, a1_ref, d2_ref, g1a_ref, g1b_ref,
                gb1_ref, g2_ref, gb2_ref, uw_ref, ub_ref, lng_ref, lnb_ref):
    ones_h1 = jnp.ones((H, 1), _f32)
    a = a0_ref[0] + a1_ref[0]
    dcol = lax.dot_general(d2_ref[...], jnp.ones((1, NWORK), _f32),
                           (((0,), (1,)), ((), ())),
                           preferred_element_type=_f32)
    inv_b = _mm_t(1.0 / (dcol + 1e-16), ones_h1)
    aggr = a * inv_b
    hpv = hp_ref[...]
    gi1 = (_mm_t(aggr, g1a_ref[...]) + _mm_t(hpv, g1b_ref[...])
           + gb1_ref[...])
    gate = jax.nn.sigmoid(_mm_t(jnp.maximum(gi1, 0.0), g2_ref[...])
                          + gb2_ref[...])
    gated = gate * aggr + (1.0 - gate) * hpv
    out = jnp.maximum(_mm_t(gated, uw_ref[...]) + ub_ref[...], 0.0)
    v = out + h_ref[...]
    ones_hh = jnp.full((H, H), 1.0 / H, _f32)
    mu_b = _mm_t(v, ones_hh)
    cen = v - mu_b
    var_b = _mm_t(cen * cen, ones_hh)
    return cen / jnp.sqrt(var_b + 1e-5) * lng_ref[...] + lnb_ref[...]


def _nku(h, hp, aggr2, den2, p, pnext):
    N = h.shape[0]
    last = pnext is None
    body = _nku_body_last if last else _nku_body_proj
    in_specs = [
        pl.BlockSpec((BN, H), lambda i: (i, 0)),
        pl.BlockSpec((BN, H), lambda i: (i, 0)),
        pl.BlockSpec((1, BN, H), lambda i: (0, i, 0)),
        pl.BlockSpec((1, BN, H), lambda i: (1, i, 0)),
        pl.BlockSpec((NWORK, BN), lambda i: (0, i)),
        pl.BlockSpec((H, H), lambda i: (0, 0)),
        pl.BlockSpec((H, H), lambda i: (0, 0)),
        pl.BlockSpec((1, H), lambda i: (0, 0)),
        pl.BlockSpec((H, H), lambda i: (0, 0)),
        pl.BlockSpec((1, H), lambda i: (0, 0)),
        pl.BlockSpec((H, H), lambda i: (0, 0)),
        pl.BlockSpec((1, H), lambda i: (0, 0)),
        pl.BlockSpec((1, H), lambda i: (0, 0)),
        pl.BlockSpec((1, H), lambda i: (0, 0)),
    ]
    out_specs = [pl.BlockSpec((BN, H), lambda i: (i, 0))]
    out_shape = [jax.ShapeDtypeStruct((N, H), _f32)]
    args = [h, hp, aggr2, aggr2, den2,
            p["gate_W1"][:, :H], p["gate_W1"][:, H:],
            p["gate_b1"].reshape(1, H),
            jnp.broadcast_to(p["gate_W2"].reshape(1, H), (H, H)),
            jnp.broadcast_to(p["gate_b2"].reshape(1, 1), (1, H)),
            p["upd_W"], p["upd_b"].reshape(1, H),
            p["ln_g"].reshape(1, H), p["ln_b"].reshape(1, H)]
    if not last:
        in_specs += [pl.BlockSpec((H, H), lambda i: (0, 0)),
                     pl.BlockSpec((1, H), lambda i: (0, 0))]
        out_specs.append(pl.BlockSpec((BN, H), lambda i: (i, 0)))
        out_shape.append(jax.ShapeDtypeStruct((N, H), _f32))
        args += [pnext["xp_W"], pnext["xp_b"].reshape(1, H)]
    res = pl.pallas_call(
        body,
        grid=(N // BN,),
        in_specs=in_specs,
        out_specs=out_specs,
        out_shape=out_shape,
    )(*args)
    return (res[0], res[1]) if not last else (res[0], None)


def _readout_body(h_ref, st_ref, wv_ref, bv_ref, wo_ref, bo_ref, out_ref,
                  feats):
    for i in range(16):
        feats[pl.ds(i, 1), :] = h_ref[pl.ds(st_ref[i], 1), :]
    vv = _mm_t(feats[...], wv_ref[...]) + bv_ref[...]
    out_ref[...] = _mm_t(vv, wo_ref[...]) + bo_ref[...]


def _readout(h, starts, params):
    N = h.shape[0]
    return pl.pallas_call(
        _readout_body,
        in_specs=[
            pl.BlockSpec((N, H), lambda: (0, 0)),
            pl.BlockSpec(memory_space=pltpu.SMEM),
            pl.BlockSpec((H, H), lambda: (0, 0)),
            pl.BlockSpec((1, H), lambda: (0, 0)),
            pl.BlockSpec((H, H), lambda: (0, 0)),
            pl.BlockSpec((1, H), lambda: (0, 0)),
        ],
        out_specs=pl.BlockSpec((16, H), lambda: (0, 0)),
        out_shape=jax.ShapeDtypeStruct((16, H), _f32),
        scratch_shapes=[pltpu.VMEM((16, H), _f32)],
    )(h, starts, params["mha_Wv"], params["mha_bv"].reshape(1, H),
      params["mha_Wo"], params["mha_bo"].reshape(1, H))


def kernel(x, pos, params, edge_index, num_aa):
    N = x.shape[0]
    E = edge_index.shape[1]
    src = edge_index[0]
    dst = edge_index[1]
    pos4 = jnp.pad(pos, ((0, 0), (0, 1)))
    x = jnp.pad(x, ((0, NP - N), (0, 0)))

    prologue = _make_prologue(E, N)
    gather_rows = _make_gather_rows(E, N)
    segmax = _make_segmax(E)
    segdenagg = _make_segdenagg(E)

    posf = pos4.reshape(N * 4)
    pef = jnp.concatenate([p["pos_emb"] for p in params["layers"]],
                          axis=1).reshape(512 * 48)
    vecf, pff = prologue(posf, src, dst, pef)
    vecT = vecf.reshape(4, E)
    pfA = pff.reshape(E, 48)
    h, hp = _nk0(x, params)
    nl = len(params["layers"])
    for l, p in enumerate(params["layers"]):
        hsrc = gather_rows(hp, src)
        t, score = _edge_pass(hsrc, vecT, pfA, p, l)
        m2 = segmax(score, dst)
        den2, aggr2 = segdenagg(m2, score, dst, t)
        pnext = params["layers"][l + 1] if l + 1 < nl else None
        h, hp = _nku(h, hp, aggr2.reshape(2, NP, H),
                     den2.reshape(NWORK, NP), p, pnext)

    starts = jnp.cumsum(num_aa) - num_aa
    return _readout(h, starts.astype(_i32), params)
